# baseline (speedup 1.0000x reference)
import functools

import jax
import jax.numpy as jnp
from jax import lax
from jax.experimental import pallas as pl
from jax.experimental.pallas import tpu as pltpu
from jax.experimental.pallas import tpu_sc as plsc

_M, _K, _KT, _DT = 1024, 32, 16, 64


def _fps_body(m, xyzT_ref, ctr_ref, d2_ref):
    b, _, n = xyzT_ref.shape
    px = xyzT_ref[:, 0, :]
    py = xyzT_ref[:, 1, :]
    pz = xyzT_ref[:, 2, :]
    lane = jax.lax.broadcasted_iota(jnp.int32, (b, n), 1)
    mlane = jax.lax.broadcasted_iota(jnp.int32, (b, m), 1)
    inf = jnp.float32(jnp.inf)
    d2_ref[...] = jnp.full((b, n), inf, jnp.float32)
    zero = jnp.zeros((b, m), jnp.float32)

    def extract(sel, p):
        return jnp.max(jnp.where(sel, p, -inf), axis=1, keepdims=True)

    def body(i, carry):
        cx, cy, cz, cur = carry
        sel = lane == cur
        lx = extract(sel, px)
        ly = extract(sel, py)
        lz = extract(sel, pz)
        rec = mlane == (i - 1)
        cx = jnp.where(rec, lx, cx)
        cy = jnp.where(rec, ly, cy)
        cz = jnp.where(rec, lz, cz)
        dx = px - lx
        dy = py - ly
        dz = pz - lz
        dist = dx * dx + dy * dy + dz * dz
        d2 = jnp.minimum(d2_ref[...], dist)
        d2_ref[...] = d2
        mx = jnp.max(d2, axis=1, keepdims=True)
        nxt = jnp.min(jnp.where(d2 == mx, lane, n), axis=1, keepdims=True)
        return cx, cy, cz, nxt

    init = (zero, zero, zero, jnp.zeros((b, 1), jnp.int32))
    cx, cy, cz, cur = jax.lax.fori_loop(1, m, body, init)
    sel = lane == cur
    rec = mlane == (m - 1)
    cx = jnp.where(rec, extract(sel, px), cx)
    cy = jnp.where(rec, extract(sel, py), cy)
    cz = jnp.where(rec, extract(sel, pz), cz)
    ctr_ref[:, 0, :] = cx
    ctr_ref[:, 1, :] = cy
    ctr_ref[:, 2, :] = cz


def _fps(xyzT, m):
    b, _, n = xyzT.shape
    return pl.pallas_call(
        functools.partial(_fps_body, m),
        out_shape=jax.ShapeDtypeStruct((b, 3, m), jnp.float32),
        scratch_shapes=[pltpu.VMEM((b, n), jnp.float32)],
    )(xyzT)


def _sc_gather(table, idx, ch=128):
    r, d = table.shape
    bsz = idx.shape[0]
    nw = 32
    b_per_w = bsz // nw
    ch = min(ch, b_per_w)
    nch = b_per_w // ch
    assert b_per_w % ch == 0 and nch % 2 == 0 or nch == 1, (bsz, ch)
    mesh = plsc.VectorSubcoreMesh(core_axis_name="c", subcore_axis_name="s")

    @functools.partial(
        pl.kernel, mesh=mesh,
        out_type=jax.ShapeDtypeStruct((bsz, d), table.dtype),
        scratch_types=[
            pltpu.VMEM((2, ch), jnp.int32),
            pltpu.VMEM((2, ch, d), table.dtype),
            pltpu.SemaphoreType.DMA,
            pltpu.SemaphoreType.DMA,
            pltpu.SemaphoreType.DMA,
            pltpu.SemaphoreType.DMA,
            pltpu.SemaphoreType.DMA,
            pltpu.SemaphoreType.DMA,
        ])
    def k(table_hbm, idx_hbm, out_hbm, idx_v, rows_v,
          si0, si1, sg0, sg1, so0, so1):
        wid = lax.axis_index("s") * 2 + lax.axis_index("c")
        base = wid * b_per_w
        si = (si0, si1)
        sg = (sg0, sg1)
        so = (so0, so1)

        if nch == 1:
            pltpu.sync_copy(idx_hbm.at[pl.ds(base, ch)], idx_v.at[0])
            pltpu.async_copy(table_hbm.at[idx_v.at[0]], rows_v.at[0],
                             sg0).wait()
            pltpu.sync_copy(rows_v.at[0], out_hbm.at[pl.ds(base, ch)])
            return

        def step(s, _):
            c0 = base + (2 * s) * ch
            c1 = c0 + ch
            cp_i0 = pltpu.async_copy(idx_hbm.at[pl.ds(c0, ch)],
                                     idx_v.at[0], si[0])
            cp_i1 = pltpu.async_copy(idx_hbm.at[pl.ds(c1, ch)],
                                     idx_v.at[1], si[1])
            cp_i0.wait()
            cp_g0 = pltpu.async_copy(table_hbm.at[idx_v.at[0]],
                                     rows_v.at[0], sg[0])
            cp_i1.wait()
            cp_g0.wait()
            cp_g1 = pltpu.async_copy(table_hbm.at[idx_v.at[1]],
                                     rows_v.at[1], sg[1])
            cp_o0 = pltpu.async_copy(rows_v.at[0],
                                     out_hbm.at[pl.ds(c0, ch)], so[0])
            cp_g1.wait()
            cp_o1 = pltpu.async_copy(rows_v.at[1],
                                     out_hbm.at[pl.ds(c1, ch)], so[1])
            cp_o0.wait()
            cp_o1.wait()
            return _

        jax.lax.fori_loop(0, nch // 2, step, 0)

    return k(table, idx)


def _flat_ids(ids, b, stride):
    off = (jnp.arange(b, dtype=jnp.int32) * stride).reshape(
        (b,) + (1,) * (ids.ndim - 1))
    return (ids + off).reshape(-1)


def _sc_gather_bf16(table, idx):
    r, d = table.shape
    t32 = jax.lax.bitcast_convert_type(
        table.reshape(r, d // 2, 2), jnp.int32)
    out = _sc_gather(t32, idx)
    return jax.lax.bitcast_convert_type(out, jnp.bfloat16).reshape(-1, d)


def _knn_body(k, q_ref, rT_ref, ids_ref, d2_ref):
    tm = q_ref.shape[1]
    n = rT_ref.shape[2]
    q = q_ref[0]
    rT = rT_ref[0]
    qq = jnp.sum(q * q, axis=1, keepdims=True)
    rr = jnp.sum(rT * rT, axis=0, keepdims=True)
    cross = jax.lax.dot_general(
        q, rT, (((1,), (0,)), ((), ())), preferred_element_type=jnp.float32)
    d2_ref[...] = qq - 2.0 * cross + rr
    lane = jax.lax.broadcasted_iota(jnp.int32, (tm, n), 1)
    klane = jax.lax.broadcasted_iota(jnp.int32, (tm, k), 1)
    inf = jnp.float32(jnp.inf)

    def body(j, ids_acc):
        d2 = d2_ref[...]
        mn = jnp.min(d2, axis=1, keepdims=True)
        am = jnp.min(jnp.where(d2 == mn, lane, n), axis=1, keepdims=True)
        ids_acc = jnp.where(klane == j, am, ids_acc)
        d2_ref[...] = jnp.where(lane == am, inf, d2)
        return ids_acc

    ids_ref[0] = jax.lax.fori_loop(
        0, k, body, jnp.zeros((tm, k), jnp.int32))


def _knn(q, rT, k, tm):
    b, mq, _ = q.shape
    n = rT.shape[2]
    return pl.pallas_call(
        functools.partial(_knn_body, k),
        grid=(b, mq // tm),
        in_specs=[
            pl.BlockSpec((1, tm, 3), lambda i, j: (i, j, 0)),
            pl.BlockSpec((1, 3, n), lambda i, j: (i, 0, 0)),
        ],
        out_specs=pl.BlockSpec((1, tm, k), lambda i, j: (i, j, 0)),
        out_shape=jax.ShapeDtypeStruct((b, mq, k), jnp.int32),
        scratch_shapes=[pltpu.VMEM((tm, n), jnp.float32)],
    )(q, rT)


def _f1_body(a_ref, w1f_ref, c_ref, w1x_ref, out_ref):
    out_ref[0] = (
        jax.lax.dot_general(a_ref[0], w1f_ref[...], (((1,), (1,)), ((), ())),
                            preferred_element_type=jnp.float32)
        + jax.lax.dot_general(c_ref[0], w1x_ref[...], (((1,), (1,)), ((), ())),
                              preferred_element_type=jnp.float32)
    ).astype(jnp.bfloat16)


def _f1(ftsT, xyz, w1f, w1x):
    b, n, c = ftsT.shape
    o = w1f.shape[0]
    return pl.pallas_call(
        _f1_body,
        grid=(b,),
        in_specs=[
            pl.BlockSpec((1, n, c), lambda i: (i, 0, 0)),
            pl.BlockSpec(w1f.shape, lambda i: (0, 0)),
            pl.BlockSpec((1, n, 3), lambda i: (i, 0, 0)),
            pl.BlockSpec(w1x.shape, lambda i: (0, 0)),
        ],
        out_specs=pl.BlockSpec((1, n, o), lambda i: (i, 0, 0)),
        out_shape=jax.ShapeDtypeStruct((b, n, o), jnp.bfloat16),
    )(ftsT, w1f, xyz, w1x)


def _mlp_body(k, f1g_ref, c_ref, w1x_ref, b1_ref, w2_ref, b2_ref, out_ref):
    rows = f1g_ref.shape[1]
    tm = rows // k
    d1 = f1g_ref.shape[2]
    cterm = jax.lax.dot_general(
        c_ref[0], w1x_ref[...], (((1,), (1,)), ((), ())),
        preferred_element_type=jnp.float32)
    crep = jnp.broadcast_to(cterm[:, None, :], (tm, k, d1)).reshape(rows, d1)
    z = jnp.maximum(f1g_ref[0].astype(jnp.float32) - crep + b1_ref[...], 0.0)
    y = jax.lax.dot_general(
        z, w2_ref[...], (((1,), (1,)), ((), ())),
        preferred_element_type=jnp.float32) + b2_ref[...]
    d2o = y.shape[1]
    out_ref[0] = jnp.max(y.reshape(tm, k, d2o), axis=1)


def _mlp(f1g, c, w1x, b1, w2, b2, k, tm):
    b, rows, d1 = f1g.shape
    m = rows // k
    d2o = w2.shape[0]
    grid = (b, m // tm)
    return pl.pallas_call(
        functools.partial(_mlp_body, k),
        grid=grid,
        in_specs=[
            pl.BlockSpec((1, tm * k, d1), lambda i, j: (i, j, 0)),
            pl.BlockSpec((1, tm, 3), lambda i, j: (i, j, 0)),
            pl.BlockSpec(w1x.shape, lambda i, j: (0, 0)),
            pl.BlockSpec(b1.shape, lambda i, j: (0, 0)),
            pl.BlockSpec(w2.shape, lambda i, j: (0, 0)),
            pl.BlockSpec(b2.shape, lambda i, j: (0, 0)),
        ],
        out_specs=pl.BlockSpec((1, tm, d2o), lambda i, j: (i, j, 0)),
        out_shape=jax.ShapeDtypeStruct((b, m, d2o), jnp.float32),
    )(f1g, c, w1x, b1, w2, b2)


def _qt2_body(x_ref, c_ref, wq_ref, wkv_ref, wp1_ref, q_ref, t2_ref):
    x = x_ref[0]
    q_ref[0] = jax.lax.dot_general(
        x, wq_ref[...], (((1,), (1,)), ((), ())),
        preferred_element_type=jnp.float32)
    kv = jax.lax.dot_general(
        x, wkv_ref[...], (((1,), (1,)), ((), ())),
        preferred_element_type=jnp.float32)
    cw = jax.lax.dot_general(
        c_ref[0], wp1_ref[...], (((1,), (1,)), ((), ())),
        preferred_element_type=jnp.float32)
    t2_ref[0] = jnp.concatenate([kv, cw, cw], axis=1).astype(jnp.bfloat16)


def _qt2(x, c, wq, wkv, wp1):
    b, m, _ = x.shape
    dt = wq.shape[0]
    return pl.pallas_call(
        _qt2_body,
        grid=(b,),
        in_specs=[
            pl.BlockSpec((1, m, x.shape[2]), lambda i: (i, 0, 0)),
            pl.BlockSpec((1, m, 3), lambda i: (i, 0, 0)),
            pl.BlockSpec(wq.shape, lambda i: (0, 0)),
            pl.BlockSpec(wkv.shape, lambda i: (0, 0)),
            pl.BlockSpec(wp1.shape, lambda i: (0, 0)),
        ],
        out_specs=[
            pl.BlockSpec((1, m, dt), lambda i: (i, 0, 0)),
            pl.BlockSpec((1, m, 4 * dt), lambda i: (i, 0, 0)),
        ],
        out_shape=[
            jax.ShapeDtypeStruct((b, m, dt), jnp.float32),
            jax.ShapeDtypeStruct((b, m, 4 * dt), jnp.bfloat16),
        ],
    )(x, c, wq, wkv, wp1)


def _attn_body(kt, q_ref, t2g_ref, cwq_ref, x_ref,
               bp1_ref, wp2_ref, bp2_ref,
               wg1_ref, bg1_ref, wg2_ref, bg2_ref,
               wo_ref, bo_ref, out_ref):
    rows = t2g_ref.shape[1]
    tm = rows // kt
    dt = q_ref.shape[2]
    t2 = t2g_ref[0].astype(jnp.float32)
    kg = t2[:, :dt]
    vg = t2[:, dt:2 * dt]
    cwg = t2[:, 2 * dt:3 * dt]
    cwq = cwq_ref[0].astype(jnp.float32)
    cwq_rep = jnp.broadcast_to(
        cwq[:, None, :], (tm, kt, dt)).reshape(rows, dt)
    delta = jnp.maximum(cwq_rep - cwg + bp1_ref[...], 0.0)
    delta = jax.lax.dot_general(
        delta, wp2_ref[...], (((1,), (1,)), ((), ())),
        preferred_element_type=jnp.float32) + bp2_ref[...]
    q = q_ref[0]
    qrep = jnp.broadcast_to(q[:, None, :], (tm, kt, dt)).reshape(rows, dt)
    a = qrep - kg + delta
    a = jax.lax.dot_general(
        a, wg1_ref[...], (((1,), (1,)), ((), ())),
        preferred_element_type=jnp.float32) + bg1_ref[...]
    a = jnp.maximum(a, 0.0)
    a = jax.lax.dot_general(
        a, wg2_ref[...], (((1,), (1,)), ((), ())),
        preferred_element_type=jnp.float32) + bg2_ref[...]
    a3 = a.reshape(tm, kt, dt)
    amax = jnp.max(a3, axis=1, keepdims=True)
    e = jnp.exp(a3 - amax)
    s = jnp.sum(e, axis=1, keepdims=True)
    attn = e / s
    vpd = (vg + delta).reshape(tm, kt, dt)
    y = jnp.sum(attn * vpd, axis=1)
    out = jax.lax.dot_general(
        y, wo_ref[...], (((1,), (1,)), ((), ())),
        preferred_element_type=jnp.float32) + bo_ref[...]
    out_ref[0] = out + x_ref[0]


def _attn(q, t2g, cwq, x, bp1, wp2, bp2, wg1, bg1, wg2, bg2, wo, bo, kt, tm):
    b, m, dt = q.shape
    do = wo.shape[0]
    grid = (b, m // tm)
    return pl.pallas_call(
        functools.partial(_attn_body, kt),
        grid=grid,
        in_specs=[
            pl.BlockSpec((1, tm, dt), lambda i, j: (i, j, 0)),
            pl.BlockSpec((1, tm * kt, 4 * dt), lambda i, j: (i, j, 0)),
            pl.BlockSpec((1, tm, dt), lambda i, j: (i, j, 0)),
            pl.BlockSpec((1, tm, do), lambda i, j: (i, j, 0)),
            pl.BlockSpec(bp1.shape, lambda i, j: (0, 0)),
            pl.BlockSpec(wp2.shape, lambda i, j: (0, 0)),
            pl.BlockSpec(bp2.shape, lambda i, j: (0, 0)),
            pl.BlockSpec(wg1.shape, lambda i, j: (0, 0)),
            pl.BlockSpec(bg1.shape, lambda i, j: (0, 0)),
            pl.BlockSpec(wg2.shape, lambda i, j: (0, 0)),
            pl.BlockSpec(bg2.shape, lambda i, j: (0, 0)),
            pl.BlockSpec(wo.shape, lambda i, j: (0, 0)),
            pl.BlockSpec(bo.shape, lambda i, j: (0, 0)),
        ],
        out_specs=pl.BlockSpec((1, tm, do), lambda i, j: (i, j, 0)),
        out_shape=jax.ShapeDtypeStruct((b, m, do), jnp.float32),
    )(q, t2g, cwq, x, bp1, wp2, bp2, wg1, bg1, wg2, bg2, wo, bo)


def kernel(xyz, fts, W1, b1, W2, b2, Wq, Wk, Wv, Wp1, bp1, Wp2, bp2,
           Wg1, bg1, Wg2, bg2, Wo, bo):
    b, n, _ = xyz.shape
    m, k, kt, dt = _M, _K, _KT, _DT
    din = fts.shape[1]
    xyzT = jnp.transpose(xyz, (0, 2, 1))

    center_xyz = jnp.transpose(_fps(xyzT, m), (0, 2, 1))

    knn_ids = _knn(center_xyz, xyzT, k, tm=256)

    ftsT = jnp.transpose(fts, (0, 2, 1))
    w1f, w1x = W1[:, :din], W1[:, din:]
    f1 = _f1(ftsT, xyz, w1f, w1x)
    f1g = _sc_gather_bf16(f1.reshape(b * n, -1), _flat_ids(knn_ids, b, n))
    f1g = f1g.reshape(b, m * k, -1)

    x = _mlp(f1g, center_xyz, w1x, b1.reshape(1, -1), W2, b2.reshape(1, -1),
             k, tm=128)

    wkv = jnp.concatenate([Wk, Wv], axis=0)
    q, t2 = _qt2(x, center_xyz, Wq, wkv, Wp1)
    cwq = t2[..., 2 * dt:3 * dt]

    cT = jnp.transpose(center_xyz, (0, 2, 1))
    nb = _knn(center_xyz, cT, kt, tm=256)

    t2g = _sc_gather_bf16(t2.reshape(b * m, -1), _flat_ids(nb, b, m))
    t2g = t2g.reshape(b, m * kt, -1)

    y = _attn(q, t2g, cwq, x,
              bp1.reshape(1, -1), Wp2, bp2.reshape(1, -1),
              Wg1, bg1.reshape(1, -1), Wg2, bg2.reshape(1, -1),
              Wo, bo.reshape(1, -1), kt, tm=256)

    center_fts = jnp.transpose(y, (0, 2, 1))
    return center_xyz, center_fts


# speedup vs baseline: 1.6974x; 1.6974x over previous
import functools

import jax
import jax.numpy as jnp
from jax import lax
from jax.experimental import pallas as pl
from jax.experimental.pallas import tpu as pltpu
from jax.experimental.pallas import tpu_sc as plsc

_M, _K, _KT, _DT = 1024, 32, 16, 64


def _fps_body(m, xyzT_ref, ctr_ref, d2_ref):
    b, _, n = xyzT_ref.shape
    px = xyzT_ref[:, 0, :]
    py = xyzT_ref[:, 1, :]
    pz = xyzT_ref[:, 2, :]
    lane = jax.lax.broadcasted_iota(jnp.int32, (b, n), 1)
    mlane = jax.lax.broadcasted_iota(jnp.int32, (b, m), 1)
    inf = jnp.float32(jnp.inf)
    d2_ref[...] = jnp.full((b, n), inf, jnp.float32)
    zero = jnp.zeros((b, m), jnp.float32)

    def extract(sel, p):
        return jnp.max(jnp.where(sel, p, -inf), axis=1, keepdims=True)

    def body(i, carry):
        cx, cy, cz, cur = carry
        sel = lane == cur
        lx = extract(sel, px)
        ly = extract(sel, py)
        lz = extract(sel, pz)
        rec = mlane == (i - 1)
        cx = jnp.where(rec, lx, cx)
        cy = jnp.where(rec, ly, cy)
        cz = jnp.where(rec, lz, cz)
        dx = px - lx
        dy = py - ly
        dz = pz - lz
        dist = dx * dx + dy * dy + dz * dz
        d2 = jnp.minimum(d2_ref[...], dist)
        d2_ref[...] = d2
        nxt = jnp.argmax(d2, axis=1).astype(jnp.int32)[:, None]
        return cx, cy, cz, nxt

    init = (zero, zero, zero, jnp.zeros((b, 1), jnp.int32))
    cx, cy, cz, cur = jax.lax.fori_loop(1, m, body, init)
    sel = lane == cur
    rec = mlane == (m - 1)
    cx = jnp.where(rec, extract(sel, px), cx)
    cy = jnp.where(rec, extract(sel, py), cy)
    cz = jnp.where(rec, extract(sel, pz), cz)
    ctr_ref[:, 0, :] = cx
    ctr_ref[:, 1, :] = cy
    ctr_ref[:, 2, :] = cz


def _fps(xyzT, m):
    b, _, n = xyzT.shape
    return pl.pallas_call(
        functools.partial(_fps_body, m),
        out_shape=jax.ShapeDtypeStruct((b, 3, m), jnp.float32),
        scratch_shapes=[pltpu.VMEM((b, n), jnp.float32)],
    )(xyzT)


def _sc_gather(table, idx, ch=128):
    r, d = table.shape
    bsz = idx.shape[0]
    nw = 32
    b_per_w = bsz // nw
    ch = min(ch, b_per_w)
    nch = b_per_w // ch
    assert b_per_w % ch == 0 and nch % 2 == 0 or nch == 1, (bsz, ch)
    mesh = plsc.VectorSubcoreMesh(core_axis_name="c", subcore_axis_name="s")

    @functools.partial(
        pl.kernel, mesh=mesh,
        out_type=jax.ShapeDtypeStruct((bsz, d), table.dtype),
        scratch_types=[
            pltpu.VMEM((2, ch), jnp.int32),
            pltpu.VMEM((2, ch, d), table.dtype),
            pltpu.SemaphoreType.DMA,
            pltpu.SemaphoreType.DMA,
            pltpu.SemaphoreType.DMA,
            pltpu.SemaphoreType.DMA,
            pltpu.SemaphoreType.DMA,
            pltpu.SemaphoreType.DMA,
        ])
    def k(table_hbm, idx_hbm, out_hbm, idx_v, rows_v,
          si0, si1, sg0, sg1, so0, so1):
        wid = lax.axis_index("s") * 2 + lax.axis_index("c")
        base = wid * b_per_w
        si = (si0, si1)
        sg = (sg0, sg1)
        so = (so0, so1)

        if nch == 1:
            pltpu.sync_copy(idx_hbm.at[pl.ds(base, ch)], idx_v.at[0])
            pltpu.async_copy(table_hbm.at[idx_v.at[0]], rows_v.at[0],
                             sg0).wait()
            pltpu.sync_copy(rows_v.at[0], out_hbm.at[pl.ds(base, ch)])
            return

        def step(s, _):
            c0 = base + (2 * s) * ch
            c1 = c0 + ch
            cp_i0 = pltpu.async_copy(idx_hbm.at[pl.ds(c0, ch)],
                                     idx_v.at[0], si[0])
            cp_i1 = pltpu.async_copy(idx_hbm.at[pl.ds(c1, ch)],
                                     idx_v.at[1], si[1])
            cp_i0.wait()
            cp_g0 = pltpu.async_copy(table_hbm.at[idx_v.at[0]],
                                     rows_v.at[0], sg[0])
            cp_i1.wait()
            cp_g0.wait()
            cp_g1 = pltpu.async_copy(table_hbm.at[idx_v.at[1]],
                                     rows_v.at[1], sg[1])
            cp_o0 = pltpu.async_copy(rows_v.at[0],
                                     out_hbm.at[pl.ds(c0, ch)], so[0])
            cp_g1.wait()
            cp_o1 = pltpu.async_copy(rows_v.at[1],
                                     out_hbm.at[pl.ds(c1, ch)], so[1])
            cp_o0.wait()
            cp_o1.wait()
            return _

        jax.lax.fori_loop(0, nch // 2, step, 0)

    return k(table, idx)


def _flat_ids(ids, b, stride):
    off = (jnp.arange(b, dtype=jnp.int32) * stride).reshape(
        (b,) + (1,) * (ids.ndim - 1))
    return (ids + off).reshape(-1)


def _knn_body(k, q_ref, rT_ref, ids_ref, d2_ref):
    tm = q_ref.shape[1]
    n = rT_ref.shape[2]
    q = q_ref[0]
    rT = rT_ref[0]
    qq = jnp.sum(q * q, axis=1, keepdims=True)
    rr = jnp.sum(rT * rT, axis=0, keepdims=True)
    cross = jax.lax.dot_general(
        q, rT, (((1,), (0,)), ((), ())), preferred_element_type=jnp.float32)
    d2_ref[...] = qq - 2.0 * cross + rr
    lane = jax.lax.broadcasted_iota(jnp.int32, (tm, n), 1)
    klane = jax.lax.broadcasted_iota(jnp.int32, (tm, k), 1)
    inf = jnp.float32(jnp.inf)

    def body(j, ids_acc):
        d2 = d2_ref[...]
        am = jnp.argmin(d2, axis=1).astype(jnp.int32)[:, None]
        ids_acc = jnp.where(klane == j, am, ids_acc)
        d2_ref[...] = jnp.where(lane == am, inf, d2)
        return ids_acc

    ids_ref[0] = jax.lax.fori_loop(
        0, k, body, jnp.zeros((tm, k), jnp.int32))


def _knn(q, rT, k, tm):
    b, mq, _ = q.shape
    n = rT.shape[2]
    return pl.pallas_call(
        functools.partial(_knn_body, k),
        grid=(b, mq // tm),
        in_specs=[
            pl.BlockSpec((1, tm, 3), lambda i, j: (i, j, 0)),
            pl.BlockSpec((1, 3, n), lambda i, j: (i, 0, 0)),
        ],
        out_specs=pl.BlockSpec((1, tm, k), lambda i, j: (i, j, 0)),
        out_shape=jax.ShapeDtypeStruct((b, mq, k), jnp.int32),
        scratch_shapes=[pltpu.VMEM((tm, n), jnp.float32)],
    )(q, rT)


def _f1_body(a_ref, w1f_ref, c_ref, w1x_ref, out_ref):
    out_ref[0] = (
        jax.lax.dot_general(a_ref[0], w1f_ref[...], (((1,), (1,)), ((), ())),
                            preferred_element_type=jnp.float32)
        + jax.lax.dot_general(c_ref[0], w1x_ref[...], (((1,), (1,)), ((), ())),
                              preferred_element_type=jnp.float32))


def _f1(ftsT, xyz, w1f, w1x):
    b, n, c = ftsT.shape
    o = w1f.shape[0]
    return pl.pallas_call(
        _f1_body,
        grid=(b,),
        in_specs=[
            pl.BlockSpec((1, n, c), lambda i: (i, 0, 0)),
            pl.BlockSpec(w1f.shape, lambda i: (0, 0)),
            pl.BlockSpec((1, n, 3), lambda i: (i, 0, 0)),
            pl.BlockSpec(w1x.shape, lambda i: (0, 0)),
        ],
        out_specs=pl.BlockSpec((1, n, o), lambda i: (i, 0, 0)),
        out_shape=jax.ShapeDtypeStruct((b, n, o), jnp.float32),
    )(ftsT, w1f, xyz, w1x)


def _mlp_body(k, f1g_ref, c_ref, w1x_ref, b1_ref, w2_ref, b2_ref, out_ref):
    rows = f1g_ref.shape[1]
    tm = rows // k
    d1 = f1g_ref.shape[2]
    cterm = jax.lax.dot_general(
        c_ref[0], w1x_ref[...], (((1,), (1,)), ((), ())),
        preferred_element_type=jnp.float32)
    crep = jnp.broadcast_to(cterm[:, None, :], (tm, k, d1)).reshape(rows, d1)
    z = jnp.maximum(f1g_ref[0].astype(jnp.float32) - crep + b1_ref[...], 0.0)
    y = jax.lax.dot_general(
        z, w2_ref[...], (((1,), (1,)), ((), ())),
        preferred_element_type=jnp.float32) + b2_ref[...]
    d2o = y.shape[1]
    out_ref[0] = jnp.max(y.reshape(tm, k, d2o), axis=1)


def _mlp(f1g, c, w1x, b1, w2, b2, k, tm):
    b, rows, d1 = f1g.shape
    m = rows // k
    d2o = w2.shape[0]
    grid = (b, m // tm)
    return pl.pallas_call(
        functools.partial(_mlp_body, k),
        grid=grid,
        in_specs=[
            pl.BlockSpec((1, tm * k, d1), lambda i, j: (i, j, 0)),
            pl.BlockSpec((1, tm, 3), lambda i, j: (i, j, 0)),
            pl.BlockSpec(w1x.shape, lambda i, j: (0, 0)),
            pl.BlockSpec(b1.shape, lambda i, j: (0, 0)),
            pl.BlockSpec(w2.shape, lambda i, j: (0, 0)),
            pl.BlockSpec(b2.shape, lambda i, j: (0, 0)),
        ],
        out_specs=pl.BlockSpec((1, tm, d2o), lambda i, j: (i, j, 0)),
        out_shape=jax.ShapeDtypeStruct((b, m, d2o), jnp.float32),
    )(f1g, c, w1x, b1, w2, b2)


def _qt2_body(x_ref, c_ref, wq_ref, wkv_ref, wp1_ref, q_ref, t2_ref):
    x = x_ref[0]
    q_ref[0] = jax.lax.dot_general(
        x, wq_ref[...], (((1,), (1,)), ((), ())),
        preferred_element_type=jnp.float32)
    kv = jax.lax.dot_general(
        x, wkv_ref[...], (((1,), (1,)), ((), ())),
        preferred_element_type=jnp.float32)
    cw = jax.lax.dot_general(
        c_ref[0], wp1_ref[...], (((1,), (1,)), ((), ())),
        preferred_element_type=jnp.float32)
    t2_ref[0] = jnp.concatenate([kv, cw, cw], axis=1)


def _qt2(x, c, wq, wkv, wp1):
    b, m, _ = x.shape
    dt = wq.shape[0]
    return pl.pallas_call(
        _qt2_body,
        grid=(b,),
        in_specs=[
            pl.BlockSpec((1, m, x.shape[2]), lambda i: (i, 0, 0)),
            pl.BlockSpec((1, m, 3), lambda i: (i, 0, 0)),
            pl.BlockSpec(wq.shape, lambda i: (0, 0)),
            pl.BlockSpec(wkv.shape, lambda i: (0, 0)),
            pl.BlockSpec(wp1.shape, lambda i: (0, 0)),
        ],
        out_specs=[
            pl.BlockSpec((1, m, dt), lambda i: (i, 0, 0)),
            pl.BlockSpec((1, m, 4 * dt), lambda i: (i, 0, 0)),
        ],
        out_shape=[
            jax.ShapeDtypeStruct((b, m, dt), jnp.float32),
            jax.ShapeDtypeStruct((b, m, 4 * dt), jnp.float32),
        ],
    )(x, c, wq, wkv, wp1)


def _attn_body(kt, q_ref, t2g_ref, cwq_ref, x_ref,
               bp1_ref, wp2_ref, bp2_ref,
               wg1_ref, bg1_ref, wg2_ref, bg2_ref,
               wo_ref, bo_ref, out_ref):
    rows = t2g_ref.shape[1]
    tm = rows // kt
    dt = q_ref.shape[2]
    t2 = t2g_ref[0].astype(jnp.float32)
    kg = t2[:, :dt]
    vg = t2[:, dt:2 * dt]
    cwg = t2[:, 2 * dt:3 * dt]
    cwq = cwq_ref[0].astype(jnp.float32)
    cwq_rep = jnp.broadcast_to(
        cwq[:, None, :], (tm, kt, dt)).reshape(rows, dt)
    delta = jnp.maximum(cwq_rep - cwg + bp1_ref[...], 0.0)
    delta = jax.lax.dot_general(
        delta, wp2_ref[...], (((1,), (1,)), ((), ())),
        preferred_element_type=jnp.float32) + bp2_ref[...]
    q = q_ref[0]
    qrep = jnp.broadcast_to(q[:, None, :], (tm, kt, dt)).reshape(rows, dt)
    a = qrep - kg + delta
    a = jax.lax.dot_general(
        a, wg1_ref[...], (((1,), (1,)), ((), ())),
        preferred_element_type=jnp.float32) + bg1_ref[...]
    a = jnp.maximum(a, 0.0)
    a = jax.lax.dot_general(
        a, wg2_ref[...], (((1,), (1,)), ((), ())),
        preferred_element_type=jnp.float32) + bg2_ref[...]
    a3 = a.reshape(tm, kt, dt)
    amax = jnp.max(a3, axis=1, keepdims=True)
    e = jnp.exp(a3 - amax)
    s = jnp.sum(e, axis=1, keepdims=True)
    attn = e / s
    vpd = (vg + delta).reshape(tm, kt, dt)
    y = jnp.sum(attn * vpd, axis=1)
    out = jax.lax.dot_general(
        y, wo_ref[...], (((1,), (1,)), ((), ())),
        preferred_element_type=jnp.float32) + bo_ref[...]
    out_ref[0] = out + x_ref[0]


def _attn(q, t2g, cwq, x, bp1, wp2, bp2, wg1, bg1, wg2, bg2, wo, bo, kt, tm):
    b, m, dt = q.shape
    do = wo.shape[0]
    grid = (b, m // tm)
    return pl.pallas_call(
        functools.partial(_attn_body, kt),
        grid=grid,
        in_specs=[
            pl.BlockSpec((1, tm, dt), lambda i, j: (i, j, 0)),
            pl.BlockSpec((1, tm * kt, 4 * dt), lambda i, j: (i, j, 0)),
            pl.BlockSpec((1, tm, dt), lambda i, j: (i, j, 0)),
            pl.BlockSpec((1, tm, do), lambda i, j: (i, j, 0)),
            pl.BlockSpec(bp1.shape, lambda i, j: (0, 0)),
            pl.BlockSpec(wp2.shape, lambda i, j: (0, 0)),
            pl.BlockSpec(bp2.shape, lambda i, j: (0, 0)),
            pl.BlockSpec(wg1.shape, lambda i, j: (0, 0)),
            pl.BlockSpec(bg1.shape, lambda i, j: (0, 0)),
            pl.BlockSpec(wg2.shape, lambda i, j: (0, 0)),
            pl.BlockSpec(bg2.shape, lambda i, j: (0, 0)),
            pl.BlockSpec(wo.shape, lambda i, j: (0, 0)),
            pl.BlockSpec(bo.shape, lambda i, j: (0, 0)),
        ],
        out_specs=pl.BlockSpec((1, tm, do), lambda i, j: (i, j, 0)),
        out_shape=jax.ShapeDtypeStruct((b, m, do), jnp.float32),
    )(q, t2g, cwq, x, bp1, wp2, bp2, wg1, bg1, wg2, bg2, wo, bo)


def kernel(xyz, fts, W1, b1, W2, b2, Wq, Wk, Wv, Wp1, bp1, Wp2, bp2,
           Wg1, bg1, Wg2, bg2, Wo, bo):
    b, n, _ = xyz.shape
    m, k, kt, dt = _M, _K, _KT, _DT
    din = fts.shape[1]
    xyzT = jnp.transpose(xyz, (0, 2, 1))

    center_xyz = jnp.transpose(_fps(xyzT, m), (0, 2, 1))

    knn_ids = _knn(center_xyz, xyzT, k, tm=256)

    ftsT = jnp.transpose(fts, (0, 2, 1))
    w1f, w1x = W1[:, :din], W1[:, din:]
    f1 = _f1(ftsT, xyz, w1f, w1x)
    f1g = _sc_gather(f1.reshape(b * n, -1), _flat_ids(knn_ids, b, n))
    f1g = f1g.reshape(b, m * k, -1)

    x = _mlp(f1g, center_xyz, w1x, b1.reshape(1, -1), W2, b2.reshape(1, -1),
             k, tm=128)

    wkv = jnp.concatenate([Wk, Wv], axis=0)
    q, t2 = _qt2(x, center_xyz, Wq, wkv, Wp1)
    cwq = t2[..., 2 * dt:3 * dt]

    cT = jnp.transpose(center_xyz, (0, 2, 1))
    nb = _knn(center_xyz, cT, kt, tm=256)

    t2g = _sc_gather(t2.reshape(b * m, -1), _flat_ids(nb, b, m))
    t2g = t2g.reshape(b, m * kt, -1)

    y = _attn(q, t2g, cwq, x,
              bp1.reshape(1, -1), Wp2, bp2.reshape(1, -1),
              Wg1, bg1.reshape(1, -1), Wg2, bg2.reshape(1, -1),
              Wo, bo.reshape(1, -1), kt, tm=256)

    center_fts = jnp.transpose(y, (0, 2, 1))
    return center_xyz, center_fts


# speedup vs baseline: 1.8377x; 1.0827x over previous
import functools

import jax
import jax.numpy as jnp
from jax import lax
from jax.experimental import pallas as pl
from jax.experimental.pallas import tpu as pltpu
from jax.experimental.pallas import tpu_sc as plsc

_M, _K, _KT, _DT = 1024, 32, 16, 64


def _fps_body(m, xyzT_ref, ctr_ref, d2_ref):
    b, _, n = xyzT_ref.shape
    px = xyzT_ref[:, 0, :]
    py = xyzT_ref[:, 1, :]
    pz = xyzT_ref[:, 2, :]
    lane = jax.lax.broadcasted_iota(jnp.int32, (b, n), 1)
    mlane = jax.lax.broadcasted_iota(jnp.int32, (b, m), 1)
    inf = jnp.float32(jnp.inf)
    d2_ref[...] = jnp.full((b, n), inf, jnp.float32)
    zero = jnp.zeros((b, m), jnp.float32)

    def extract(sel, p):
        return jnp.max(jnp.where(sel, p, -inf), axis=1, keepdims=True)

    def body(i, carry):
        cx, cy, cz, cur = carry
        sel = lane == cur
        lx = extract(sel, px)
        ly = extract(sel, py)
        lz = extract(sel, pz)
        rec = mlane == (i - 1)
        cx = jnp.where(rec, lx, cx)
        cy = jnp.where(rec, ly, cy)
        cz = jnp.where(rec, lz, cz)
        dx = px - lx
        dy = py - ly
        dz = pz - lz
        dist = dx * dx + dy * dy + dz * dz
        d2 = jnp.minimum(d2_ref[...], dist)
        d2_ref[...] = d2
        nxt = jnp.argmax(d2, axis=1).astype(jnp.int32)[:, None]
        return cx, cy, cz, nxt

    init = (zero, zero, zero, jnp.zeros((b, 1), jnp.int32))
    cx, cy, cz, cur = jax.lax.fori_loop(1, m, body, init)
    sel = lane == cur
    rec = mlane == (m - 1)
    cx = jnp.where(rec, extract(sel, px), cx)
    cy = jnp.where(rec, extract(sel, py), cy)
    cz = jnp.where(rec, extract(sel, pz), cz)
    ctr_ref[:, 0, :] = cx
    ctr_ref[:, 1, :] = cy
    ctr_ref[:, 2, :] = cz


def _fps(xyzT, m):
    b, _, n = xyzT.shape
    return pl.pallas_call(
        functools.partial(_fps_body, m),
        out_shape=jax.ShapeDtypeStruct((b, 3, m), jnp.float32),
        scratch_shapes=[pltpu.VMEM((b, n), jnp.float32)],
    )(xyzT)


def _sc_gather(table, idx, ch=128):
    r, d = table.shape
    bsz = idx.shape[0]
    nw = 32
    b_per_w = bsz // nw
    ch = min(ch, b_per_w)
    nch = b_per_w // ch
    assert b_per_w % ch == 0 and nch % 2 == 0 or nch == 1, (bsz, ch)
    mesh = plsc.VectorSubcoreMesh(core_axis_name="c", subcore_axis_name="s")

    @functools.partial(
        pl.kernel, mesh=mesh,
        out_type=jax.ShapeDtypeStruct((bsz, d), table.dtype),
        scratch_types=[
            pltpu.VMEM((2, ch), jnp.int32),
            pltpu.VMEM((2, ch, d), table.dtype),
            pltpu.SemaphoreType.DMA,
            pltpu.SemaphoreType.DMA,
            pltpu.SemaphoreType.DMA,
            pltpu.SemaphoreType.DMA,
            pltpu.SemaphoreType.DMA,
            pltpu.SemaphoreType.DMA,
        ])
    def k(table_hbm, idx_hbm, out_hbm, idx_v, rows_v,
          si0, si1, sg0, sg1, so0, so1):
        wid = lax.axis_index("s") * 2 + lax.axis_index("c")
        base = wid * b_per_w
        si = (si0, si1)
        sg = (sg0, sg1)
        so = (so0, so1)

        if nch == 1:
            pltpu.sync_copy(idx_hbm.at[pl.ds(base, ch)], idx_v.at[0])
            pltpu.async_copy(table_hbm.at[idx_v.at[0]], rows_v.at[0],
                             sg0).wait()
            pltpu.sync_copy(rows_v.at[0], out_hbm.at[pl.ds(base, ch)])
            return

        def step(s, _):
            c0 = base + (2 * s) * ch
            c1 = c0 + ch
            cp_i0 = pltpu.async_copy(idx_hbm.at[pl.ds(c0, ch)],
                                     idx_v.at[0], si[0])
            cp_i1 = pltpu.async_copy(idx_hbm.at[pl.ds(c1, ch)],
                                     idx_v.at[1], si[1])
            cp_i0.wait()
            cp_g0 = pltpu.async_copy(table_hbm.at[idx_v.at[0]],
                                     rows_v.at[0], sg[0])
            cp_i1.wait()
            cp_g0.wait()
            cp_g1 = pltpu.async_copy(table_hbm.at[idx_v.at[1]],
                                     rows_v.at[1], sg[1])
            cp_o0 = pltpu.async_copy(rows_v.at[0],
                                     out_hbm.at[pl.ds(c0, ch)], so[0])
            cp_g1.wait()
            cp_o1 = pltpu.async_copy(rows_v.at[1],
                                     out_hbm.at[pl.ds(c1, ch)], so[1])
            cp_o0.wait()
            cp_o1.wait()
            return _

        jax.lax.fori_loop(0, nch // 2, step, 0)

    return k(table, idx)


def _flat_ids(ids, b, stride):
    off = (jnp.arange(b, dtype=jnp.int32) * stride).reshape(
        (b,) + (1,) * (ids.ndim - 1))
    return (ids + off).reshape(-1)


def _knn_body(k, q_ref, rT_ref, ids_ref, d2_ref):
    tm = q_ref.shape[1]
    n = rT_ref.shape[2]
    q = q_ref[0]
    rT = rT_ref[0]
    qq = jnp.sum(q * q, axis=1, keepdims=True)
    rr = jnp.sum(rT * rT, axis=0, keepdims=True)
    cross = jax.lax.dot_general(
        q, rT, (((1,), (0,)), ((), ())), preferred_element_type=jnp.float32)
    d2_ref[...] = qq - 2.0 * cross + rr
    lane = jax.lax.broadcasted_iota(jnp.int32, (tm, n), 1)
    klane = jax.lax.broadcasted_iota(jnp.int32, (tm, k), 1)
    inf = jnp.float32(jnp.inf)

    def body(j, ids_acc):
        d2 = d2_ref[...]
        mn = jnp.min(d2, axis=1, keepdims=True)
        am = jnp.min(jnp.where(d2 == mn, lane, n), axis=1, keepdims=True)
        ids_acc = jnp.where(klane == j, am, ids_acc)
        d2_ref[...] = jnp.where(lane == am, inf, d2)
        return ids_acc

    ids_ref[0] = jax.lax.fori_loop(
        0, k, body, jnp.zeros((tm, k), jnp.int32))


def _knn(q, rT, k, tm):
    b, mq, _ = q.shape
    n = rT.shape[2]
    return pl.pallas_call(
        functools.partial(_knn_body, k),
        grid=(b, mq // tm),
        in_specs=[
            pl.BlockSpec((1, tm, 3), lambda i, j: (i, j, 0)),
            pl.BlockSpec((1, 3, n), lambda i, j: (i, 0, 0)),
        ],
        out_specs=pl.BlockSpec((1, tm, k), lambda i, j: (i, j, 0)),
        out_shape=jax.ShapeDtypeStruct((b, mq, k), jnp.int32),
        scratch_shapes=[pltpu.VMEM((tm, n), jnp.float32)],
    )(q, rT)


def _f1_body(a_ref, w1f_ref, c_ref, w1x_ref, out_ref):
    out_ref[0] = (
        jax.lax.dot_general(a_ref[0], w1f_ref[...], (((1,), (1,)), ((), ())),
                            preferred_element_type=jnp.float32)
        + jax.lax.dot_general(c_ref[0], w1x_ref[...], (((1,), (1,)), ((), ())),
                              preferred_element_type=jnp.float32))


def _f1(ftsT, xyz, w1f, w1x):
    b, n, c = ftsT.shape
    o = w1f.shape[0]
    return pl.pallas_call(
        _f1_body,
        grid=(b,),
        in_specs=[
            pl.BlockSpec((1, n, c), lambda i: (i, 0, 0)),
            pl.BlockSpec(w1f.shape, lambda i: (0, 0)),
            pl.BlockSpec((1, n, 3), lambda i: (i, 0, 0)),
            pl.BlockSpec(w1x.shape, lambda i: (0, 0)),
        ],
        out_specs=pl.BlockSpec((1, n, o), lambda i: (i, 0, 0)),
        out_shape=jax.ShapeDtypeStruct((b, n, o), jnp.float32),
    )(ftsT, w1f, xyz, w1x)


def _mlp_body(k, f1g_ref, c_ref, w1x_ref, b1_ref, w2_ref, b2_ref, out_ref):
    rows = f1g_ref.shape[1]
    tm = rows // k
    d1 = f1g_ref.shape[2]
    cterm = jax.lax.dot_general(
        c_ref[0], w1x_ref[...], (((1,), (1,)), ((), ())),
        preferred_element_type=jnp.float32)
    crep = jnp.broadcast_to(cterm[:, None, :], (tm, k, d1)).reshape(rows, d1)
    z = jnp.maximum(f1g_ref[0].astype(jnp.float32) - crep + b1_ref[...], 0.0)
    y = jax.lax.dot_general(
        z, w2_ref[...], (((1,), (1,)), ((), ())),
        preferred_element_type=jnp.float32) + b2_ref[...]
    d2o = y.shape[1]
    out_ref[0] = jnp.max(y.reshape(tm, k, d2o), axis=1)


def _mlp(f1g, c, w1x, b1, w2, b2, k, tm):
    b, rows, d1 = f1g.shape
    m = rows // k
    d2o = w2.shape[0]
    grid = (b, m // tm)
    return pl.pallas_call(
        functools.partial(_mlp_body, k),
        grid=grid,
        in_specs=[
            pl.BlockSpec((1, tm * k, d1), lambda i, j: (i, j, 0)),
            pl.BlockSpec((1, tm, 3), lambda i, j: (i, j, 0)),
            pl.BlockSpec(w1x.shape, lambda i, j: (0, 0)),
            pl.BlockSpec(b1.shape, lambda i, j: (0, 0)),
            pl.BlockSpec(w2.shape, lambda i, j: (0, 0)),
            pl.BlockSpec(b2.shape, lambda i, j: (0, 0)),
        ],
        out_specs=pl.BlockSpec((1, tm, d2o), lambda i, j: (i, j, 0)),
        out_shape=jax.ShapeDtypeStruct((b, m, d2o), jnp.float32),
    )(f1g, c, w1x, b1, w2, b2)


def _qt2_body(x_ref, c_ref, wq_ref, wkv_ref, wp1_ref, q_ref, t2_ref):
    x = x_ref[0]
    q_ref[0] = jax.lax.dot_general(
        x, wq_ref[...], (((1,), (1,)), ((), ())),
        preferred_element_type=jnp.float32)
    kv = jax.lax.dot_general(
        x, wkv_ref[...], (((1,), (1,)), ((), ())),
        preferred_element_type=jnp.float32)
    cw = jax.lax.dot_general(
        c_ref[0], wp1_ref[...], (((1,), (1,)), ((), ())),
        preferred_element_type=jnp.float32)
    t2_ref[0] = jnp.concatenate([kv, cw, cw], axis=1)


def _qt2(x, c, wq, wkv, wp1):
    b, m, _ = x.shape
    dt = wq.shape[0]
    return pl.pallas_call(
        _qt2_body,
        grid=(b,),
        in_specs=[
            pl.BlockSpec((1, m, x.shape[2]), lambda i: (i, 0, 0)),
            pl.BlockSpec((1, m, 3), lambda i: (i, 0, 0)),
            pl.BlockSpec(wq.shape, lambda i: (0, 0)),
            pl.BlockSpec(wkv.shape, lambda i: (0, 0)),
            pl.BlockSpec(wp1.shape, lambda i: (0, 0)),
        ],
        out_specs=[
            pl.BlockSpec((1, m, dt), lambda i: (i, 0, 0)),
            pl.BlockSpec((1, m, 4 * dt), lambda i: (i, 0, 0)),
        ],
        out_shape=[
            jax.ShapeDtypeStruct((b, m, dt), jnp.float32),
            jax.ShapeDtypeStruct((b, m, 4 * dt), jnp.float32),
        ],
    )(x, c, wq, wkv, wp1)


def _attn_body(kt, q_ref, t2g_ref, cwq_ref, x_ref,
               bp1_ref, wp2_ref, bp2_ref,
               wg1_ref, bg1_ref, wg2_ref, bg2_ref,
               wo_ref, bo_ref, out_ref):
    rows = t2g_ref.shape[1]
    tm = rows // kt
    dt = q_ref.shape[2]
    t2 = t2g_ref[0].astype(jnp.float32)
    kg = t2[:, :dt]
    vg = t2[:, dt:2 * dt]
    cwg = t2[:, 2 * dt:3 * dt]
    cwq = cwq_ref[0].astype(jnp.float32)
    cwq_rep = jnp.broadcast_to(
        cwq[:, None, :], (tm, kt, dt)).reshape(rows, dt)
    delta = jnp.maximum(cwq_rep - cwg + bp1_ref[...], 0.0)
    delta = jax.lax.dot_general(
        delta, wp2_ref[...], (((1,), (1,)), ((), ())),
        preferred_element_type=jnp.float32) + bp2_ref[...]
    q = q_ref[0]
    qrep = jnp.broadcast_to(q[:, None, :], (tm, kt, dt)).reshape(rows, dt)
    a = qrep - kg + delta
    a = jax.lax.dot_general(
        a, wg1_ref[...], (((1,), (1,)), ((), ())),
        preferred_element_type=jnp.float32) + bg1_ref[...]
    a = jnp.maximum(a, 0.0)
    a = jax.lax.dot_general(
        a, wg2_ref[...], (((1,), (1,)), ((), ())),
        preferred_element_type=jnp.float32) + bg2_ref[...]
    a3 = a.reshape(tm, kt, dt)
    amax = jnp.max(a3, axis=1, keepdims=True)
    e = jnp.exp(a3 - amax)
    s = jnp.sum(e, axis=1, keepdims=True)
    attn = e / s
    vpd = (vg + delta).reshape(tm, kt, dt)
    y = jnp.sum(attn * vpd, axis=1)
    out = jax.lax.dot_general(
        y, wo_ref[...], (((1,), (1,)), ((), ())),
        preferred_element_type=jnp.float32) + bo_ref[...]
    out_ref[0] = out + x_ref[0]


def _attn(q, t2g, cwq, x, bp1, wp2, bp2, wg1, bg1, wg2, bg2, wo, bo, kt, tm):
    b, m, dt = q.shape
    do = wo.shape[0]
    grid = (b, m // tm)
    return pl.pallas_call(
        functools.partial(_attn_body, kt),
        grid=grid,
        in_specs=[
            pl.BlockSpec((1, tm, dt), lambda i, j: (i, j, 0)),
            pl.BlockSpec((1, tm * kt, 4 * dt), lambda i, j: (i, j, 0)),
            pl.BlockSpec((1, tm, dt), lambda i, j: (i, j, 0)),
            pl.BlockSpec((1, tm, do), lambda i, j: (i, j, 0)),
            pl.BlockSpec(bp1.shape, lambda i, j: (0, 0)),
            pl.BlockSpec(wp2.shape, lambda i, j: (0, 0)),
            pl.BlockSpec(bp2.shape, lambda i, j: (0, 0)),
            pl.BlockSpec(wg1.shape, lambda i, j: (0, 0)),
            pl.BlockSpec(bg1.shape, lambda i, j: (0, 0)),
            pl.BlockSpec(wg2.shape, lambda i, j: (0, 0)),
            pl.BlockSpec(bg2.shape, lambda i, j: (0, 0)),
            pl.BlockSpec(wo.shape, lambda i, j: (0, 0)),
            pl.BlockSpec(bo.shape, lambda i, j: (0, 0)),
        ],
        out_specs=pl.BlockSpec((1, tm, do), lambda i, j: (i, j, 0)),
        out_shape=jax.ShapeDtypeStruct((b, m, do), jnp.float32),
    )(q, t2g, cwq, x, bp1, wp2, bp2, wg1, bg1, wg2, bg2, wo, bo)


def kernel(xyz, fts, W1, b1, W2, b2, Wq, Wk, Wv, Wp1, bp1, Wp2, bp2,
           Wg1, bg1, Wg2, bg2, Wo, bo):
    b, n, _ = xyz.shape
    m, k, kt, dt = _M, _K, _KT, _DT
    din = fts.shape[1]
    xyzT = jnp.transpose(xyz, (0, 2, 1))

    center_xyz = jnp.transpose(_fps(xyzT, m), (0, 2, 1))

    knn_ids = _knn(center_xyz, xyzT, k, tm=256)

    ftsT = jnp.transpose(fts, (0, 2, 1))
    w1f, w1x = W1[:, :din], W1[:, din:]
    f1 = _f1(ftsT, xyz, w1f, w1x)
    f1g = _sc_gather(f1.reshape(b * n, -1), _flat_ids(knn_ids, b, n))
    f1g = f1g.reshape(b, m * k, -1)

    x = _mlp(f1g, center_xyz, w1x, b1.reshape(1, -1), W2, b2.reshape(1, -1),
             k, tm=128)

    wkv = jnp.concatenate([Wk, Wv], axis=0)
    q, t2 = _qt2(x, center_xyz, Wq, wkv, Wp1)
    cwq = t2[..., 2 * dt:3 * dt]

    cT = jnp.transpose(center_xyz, (0, 2, 1))
    nb = _knn(center_xyz, cT, kt, tm=256)

    t2g = _sc_gather(t2.reshape(b * m, -1), _flat_ids(nb, b, m))
    t2g = t2g.reshape(b, m * kt, -1)

    y = _attn(q, t2g, cwq, x,
              bp1.reshape(1, -1), Wp2, bp2.reshape(1, -1),
              Wg1, bg1.reshape(1, -1), Wg2, bg2.reshape(1, -1),
              Wo, bo.reshape(1, -1), kt, tm=256)

    center_fts = jnp.transpose(y, (0, 2, 1))
    return center_xyz, center_fts


# speedup vs baseline: 1.8983x; 1.0330x over previous
import functools

import jax
import jax.numpy as jnp
from jax import lax
from jax.experimental import pallas as pl
from jax.experimental.pallas import tpu as pltpu
from jax.experimental.pallas import tpu_sc as plsc

_M, _K, _KT, _DT = 1024, 32, 16, 64


def _fps_body(m, xyz3_ref, ctr_ref, d2_ref):
    _, b, n = xyz3_ref.shape
    s = xyz3_ref[...].reshape(3 * b, n)
    lane3 = jax.lax.broadcasted_iota(jnp.int32, (3 * b, n), 1)
    mlane = jax.lax.broadcasted_iota(jnp.int32, (b, m), 1)
    inf = jnp.float32(jnp.inf)
    d2_ref[...] = jnp.full((b, n), inf, jnp.float32)
    zero = jnp.zeros((b, m), jnp.float32)

    def extract(cur):
        sel3 = lane3 == jnp.broadcast_to(cur[None], (3, b, 1)).reshape(3 * b, 1)
        return jnp.max(jnp.where(sel3, s, -inf), axis=1, keepdims=True)

    def body(i, carry):
        cx, cy, cz, cur = carry
        l3 = extract(cur)
        rec = mlane == (i - 1)
        cx = jnp.where(rec, l3[:b], cx)
        cy = jnp.where(rec, l3[b:2 * b], cy)
        cz = jnp.where(rec, l3[2 * b:], cz)
        d3 = s - l3
        sq = d3 * d3
        dist = sq[:b] + sq[b:2 * b] + sq[2 * b:]
        d2 = jnp.minimum(d2_ref[...], dist)
        d2_ref[...] = d2
        nxt = jnp.argmax(d2, axis=1).astype(jnp.int32)[:, None]
        return cx, cy, cz, nxt

    init = (zero, zero, zero, jnp.zeros((b, 1), jnp.int32))
    cx, cy, cz, cur = jax.lax.fori_loop(1, m, body, init)
    l3 = extract(cur)
    rec = mlane == (m - 1)
    ctr_ref[:, 0, :] = jnp.where(rec, l3[:b], cx)
    ctr_ref[:, 1, :] = jnp.where(rec, l3[b:2 * b], cy)
    ctr_ref[:, 2, :] = jnp.where(rec, l3[2 * b:], cz)


def _fps(xyz3, m):
    _, b, n = xyz3.shape
    return pl.pallas_call(
        functools.partial(_fps_body, m),
        out_shape=jax.ShapeDtypeStruct((b, 3, m), jnp.float32),
        scratch_shapes=[pltpu.VMEM((b, n), jnp.float32)],
    )(xyz3)


def _sc_gather(table, idx, ch=128):
    r, d = table.shape
    bsz = idx.shape[0]
    nw = 32
    b_per_w = bsz // nw
    ch = min(ch, b_per_w)
    nch = b_per_w // ch
    assert b_per_w % ch == 0 and nch % 2 == 0 or nch == 1, (bsz, ch)
    mesh = plsc.VectorSubcoreMesh(core_axis_name="c", subcore_axis_name="s")

    @functools.partial(
        pl.kernel, mesh=mesh,
        out_type=jax.ShapeDtypeStruct((bsz, d), table.dtype),
        scratch_types=[
            pltpu.VMEM((2, ch), jnp.int32),
            pltpu.VMEM((2, ch, d), table.dtype),
            pltpu.SemaphoreType.DMA,
            pltpu.SemaphoreType.DMA,
            pltpu.SemaphoreType.DMA,
            pltpu.SemaphoreType.DMA,
            pltpu.SemaphoreType.DMA,
            pltpu.SemaphoreType.DMA,
        ])
    def k(table_hbm, idx_hbm, out_hbm, idx_v, rows_v,
          si0, si1, sg0, sg1, so0, so1):
        wid = lax.axis_index("s") * 2 + lax.axis_index("c")
        base = wid * b_per_w
        si = (si0, si1)
        sg = (sg0, sg1)
        so = (so0, so1)

        if nch == 1:
            pltpu.sync_copy(idx_hbm.at[pl.ds(base, ch)], idx_v.at[0])
            pltpu.async_copy(table_hbm.at[idx_v.at[0]], rows_v.at[0],
                             sg0).wait()
            pltpu.sync_copy(rows_v.at[0], out_hbm.at[pl.ds(base, ch)])
            return

        def step(s, _):
            c0 = base + (2 * s) * ch
            c1 = c0 + ch
            cp_i0 = pltpu.async_copy(idx_hbm.at[pl.ds(c0, ch)],
                                     idx_v.at[0], si[0])
            cp_i1 = pltpu.async_copy(idx_hbm.at[pl.ds(c1, ch)],
                                     idx_v.at[1], si[1])
            cp_i0.wait()
            cp_g0 = pltpu.async_copy(table_hbm.at[idx_v.at[0]],
                                     rows_v.at[0], sg[0])
            cp_i1.wait()
            cp_g0.wait()
            cp_g1 = pltpu.async_copy(table_hbm.at[idx_v.at[1]],
                                     rows_v.at[1], sg[1])
            cp_o0 = pltpu.async_copy(rows_v.at[0],
                                     out_hbm.at[pl.ds(c0, ch)], so[0])
            cp_g1.wait()
            cp_o1 = pltpu.async_copy(rows_v.at[1],
                                     out_hbm.at[pl.ds(c1, ch)], so[1])
            cp_o0.wait()
            cp_o1.wait()
            return _

        jax.lax.fori_loop(0, nch // 2, step, 0)

    return k(table, idx)


def _flat_ids(ids, b, stride):
    off = (jnp.arange(b, dtype=jnp.int32) * stride).reshape(
        (b,) + (1,) * (ids.ndim - 1))
    return (ids + off).reshape(-1)


def _knn_body(k, q_ref, rT_ref, ids_ref, d2_ref):
    tm = q_ref.shape[1]
    n = rT_ref.shape[2]
    q = q_ref[0]
    rT = rT_ref[0]
    qq = jnp.sum(q * q, axis=1, keepdims=True)
    rr = jnp.sum(rT * rT, axis=0, keepdims=True)
    cross = jax.lax.dot_general(
        q, rT, (((1,), (0,)), ((), ())), preferred_element_type=jnp.float32)
    d2_ref[...] = qq - 2.0 * cross + rr
    lane = jax.lax.broadcasted_iota(jnp.int32, (tm, n), 1)
    klane = jax.lax.broadcasted_iota(jnp.int32, (tm, k), 1)
    inf = jnp.float32(jnp.inf)

    def body(j, ids_acc):
        d2 = d2_ref[...]
        mn = jnp.min(d2, axis=1, keepdims=True)
        am = jnp.min(jnp.where(d2 == mn, lane, n), axis=1, keepdims=True)
        ids_acc = jnp.where(klane == j, am, ids_acc)
        d2_ref[...] = jnp.where(lane == am, inf, d2)
        return ids_acc

    ids_ref[0] = jax.lax.fori_loop(
        0, k, body, jnp.zeros((tm, k), jnp.int32))


def _knn(q, rT, k, tm):
    b, mq, _ = q.shape
    n = rT.shape[2]
    return pl.pallas_call(
        functools.partial(_knn_body, k),
        grid=(b, mq // tm),
        in_specs=[
            pl.BlockSpec((1, tm, 3), lambda i, j: (i, j, 0)),
            pl.BlockSpec((1, 3, n), lambda i, j: (i, 0, 0)),
        ],
        out_specs=pl.BlockSpec((1, tm, k), lambda i, j: (i, j, 0)),
        out_shape=jax.ShapeDtypeStruct((b, mq, k), jnp.int32),
        scratch_shapes=[pltpu.VMEM((tm, n), jnp.float32)],
    )(q, rT)


def _f1_body(a_ref, w1f_ref, c_ref, w1x_ref, out_ref):
    out_ref[0] = (
        jax.lax.dot_general(a_ref[0], w1f_ref[...], (((1,), (1,)), ((), ())),
                            preferred_element_type=jnp.float32)
        + jax.lax.dot_general(c_ref[0], w1x_ref[...], (((1,), (1,)), ((), ())),
                              preferred_element_type=jnp.float32))


def _f1(ftsT, xyz, w1f, w1x):
    b, n, c = ftsT.shape
    o = w1f.shape[0]
    return pl.pallas_call(
        _f1_body,
        grid=(b,),
        in_specs=[
            pl.BlockSpec((1, n, c), lambda i: (i, 0, 0)),
            pl.BlockSpec(w1f.shape, lambda i: (0, 0)),
            pl.BlockSpec((1, n, 3), lambda i: (i, 0, 0)),
            pl.BlockSpec(w1x.shape, lambda i: (0, 0)),
        ],
        out_specs=pl.BlockSpec((1, n, o), lambda i: (i, 0, 0)),
        out_shape=jax.ShapeDtypeStruct((b, n, o), jnp.float32),
    )(ftsT, w1f, xyz, w1x)


def _mlp_body(k, f1g_ref, c_ref, w1x_ref, b1_ref, w2_ref, b2_ref, out_ref):
    rows = f1g_ref.shape[1]
    tm = rows // k
    d1 = f1g_ref.shape[2]
    cterm = jax.lax.dot_general(
        c_ref[0], w1x_ref[...], (((1,), (1,)), ((), ())),
        preferred_element_type=jnp.float32)
    crep = jnp.broadcast_to(cterm[:, None, :], (tm, k, d1)).reshape(rows, d1)
    z = jnp.maximum(f1g_ref[0].astype(jnp.float32) - crep + b1_ref[...], 0.0)
    y = jax.lax.dot_general(
        z, w2_ref[...], (((1,), (1,)), ((), ())),
        preferred_element_type=jnp.float32) + b2_ref[...]
    d2o = y.shape[1]
    out_ref[0] = jnp.max(y.reshape(tm, k, d2o), axis=1)


def _mlp(f1g, c, w1x, b1, w2, b2, k, tm):
    b, rows, d1 = f1g.shape
    m = rows // k
    d2o = w2.shape[0]
    grid = (b, m // tm)
    return pl.pallas_call(
        functools.partial(_mlp_body, k),
        grid=grid,
        in_specs=[
            pl.BlockSpec((1, tm * k, d1), lambda i, j: (i, j, 0)),
            pl.BlockSpec((1, tm, 3), lambda i, j: (i, j, 0)),
            pl.BlockSpec(w1x.shape, lambda i, j: (0, 0)),
            pl.BlockSpec(b1.shape, lambda i, j: (0, 0)),
            pl.BlockSpec(w2.shape, lambda i, j: (0, 0)),
            pl.BlockSpec(b2.shape, lambda i, j: (0, 0)),
        ],
        out_specs=pl.BlockSpec((1, tm, d2o), lambda i, j: (i, j, 0)),
        out_shape=jax.ShapeDtypeStruct((b, m, d2o), jnp.float32),
    )(f1g, c, w1x, b1, w2, b2)


def _qt2_body(x_ref, c_ref, wq_ref, wkv_ref, wp1_ref, q_ref, t2_ref):
    x = x_ref[0]
    q_ref[0] = jax.lax.dot_general(
        x, wq_ref[...], (((1,), (1,)), ((), ())),
        preferred_element_type=jnp.float32)
    kv = jax.lax.dot_general(
        x, wkv_ref[...], (((1,), (1,)), ((), ())),
        preferred_element_type=jnp.float32)
    cw = jax.lax.dot_general(
        c_ref[0], wp1_ref[...], (((1,), (1,)), ((), ())),
        preferred_element_type=jnp.float32)
    t2_ref[0] = jnp.concatenate([kv, cw, cw], axis=1)


def _qt2(x, c, wq, wkv, wp1):
    b, m, _ = x.shape
    dt = wq.shape[0]
    return pl.pallas_call(
        _qt2_body,
        grid=(b,),
        in_specs=[
            pl.BlockSpec((1, m, x.shape[2]), lambda i: (i, 0, 0)),
            pl.BlockSpec((1, m, 3), lambda i: (i, 0, 0)),
            pl.BlockSpec(wq.shape, lambda i: (0, 0)),
            pl.BlockSpec(wkv.shape, lambda i: (0, 0)),
            pl.BlockSpec(wp1.shape, lambda i: (0, 0)),
        ],
        out_specs=[
            pl.BlockSpec((1, m, dt), lambda i: (i, 0, 0)),
            pl.BlockSpec((1, m, 4 * dt), lambda i: (i, 0, 0)),
        ],
        out_shape=[
            jax.ShapeDtypeStruct((b, m, dt), jnp.float32),
            jax.ShapeDtypeStruct((b, m, 4 * dt), jnp.float32),
        ],
    )(x, c, wq, wkv, wp1)


def _attn_body(kt, q_ref, t2g_ref, cwq_ref, x_ref,
               bp1_ref, wp2_ref, bp2_ref,
               wg1_ref, bg1_ref, wg2_ref, bg2_ref,
               wo_ref, bo_ref, out_ref):
    rows = t2g_ref.shape[1]
    tm = rows // kt
    dt = q_ref.shape[2]
    t2 = t2g_ref[0].astype(jnp.float32)
    kg = t2[:, :dt]
    vg = t2[:, dt:2 * dt]
    cwg = t2[:, 2 * dt:3 * dt]
    cwq = cwq_ref[0].astype(jnp.float32)
    cwq_rep = jnp.broadcast_to(
        cwq[:, None, :], (tm, kt, dt)).reshape(rows, dt)
    delta = jnp.maximum(cwq_rep - cwg + bp1_ref[...], 0.0)
    delta = jax.lax.dot_general(
        delta, wp2_ref[...], (((1,), (1,)), ((), ())),
        preferred_element_type=jnp.float32) + bp2_ref[...]
    q = q_ref[0]
    qrep = jnp.broadcast_to(q[:, None, :], (tm, kt, dt)).reshape(rows, dt)
    a = qrep - kg + delta
    a = jax.lax.dot_general(
        a, wg1_ref[...], (((1,), (1,)), ((), ())),
        preferred_element_type=jnp.float32) + bg1_ref[...]
    a = jnp.maximum(a, 0.0)
    a = jax.lax.dot_general(
        a, wg2_ref[...], (((1,), (1,)), ((), ())),
        preferred_element_type=jnp.float32) + bg2_ref[...]
    a3 = a.reshape(tm, kt, dt)
    amax = jnp.max(a3, axis=1, keepdims=True)
    e = jnp.exp(a3 - amax)
    s = jnp.sum(e, axis=1, keepdims=True)
    attn = e / s
    vpd = (vg + delta).reshape(tm, kt, dt)
    y = jnp.sum(attn * vpd, axis=1)
    out = jax.lax.dot_general(
        y, wo_ref[...], (((1,), (1,)), ((), ())),
        preferred_element_type=jnp.float32) + bo_ref[...]
    out_ref[0] = out + x_ref[0]


def _attn(q, t2g, cwq, x, bp1, wp2, bp2, wg1, bg1, wg2, bg2, wo, bo, kt, tm):
    b, m, dt = q.shape
    do = wo.shape[0]
    grid = (b, m // tm)
    return pl.pallas_call(
        functools.partial(_attn_body, kt),
        grid=grid,
        in_specs=[
            pl.BlockSpec((1, tm, dt), lambda i, j: (i, j, 0)),
            pl.BlockSpec((1, tm * kt, 4 * dt), lambda i, j: (i, j, 0)),
            pl.BlockSpec((1, tm, dt), lambda i, j: (i, j, 0)),
            pl.BlockSpec((1, tm, do), lambda i, j: (i, j, 0)),
            pl.BlockSpec(bp1.shape, lambda i, j: (0, 0)),
            pl.BlockSpec(wp2.shape, lambda i, j: (0, 0)),
            pl.BlockSpec(bp2.shape, lambda i, j: (0, 0)),
            pl.BlockSpec(wg1.shape, lambda i, j: (0, 0)),
            pl.BlockSpec(bg1.shape, lambda i, j: (0, 0)),
            pl.BlockSpec(wg2.shape, lambda i, j: (0, 0)),
            pl.BlockSpec(bg2.shape, lambda i, j: (0, 0)),
            pl.BlockSpec(wo.shape, lambda i, j: (0, 0)),
            pl.BlockSpec(bo.shape, lambda i, j: (0, 0)),
        ],
        out_specs=pl.BlockSpec((1, tm, do), lambda i, j: (i, j, 0)),
        out_shape=jax.ShapeDtypeStruct((b, m, do), jnp.float32),
    )(q, t2g, cwq, x, bp1, wp2, bp2, wg1, bg1, wg2, bg2, wo, bo)


def kernel(xyz, fts, W1, b1, W2, b2, Wq, Wk, Wv, Wp1, bp1, Wp2, bp2,
           Wg1, bg1, Wg2, bg2, Wo, bo):
    b, n, _ = xyz.shape
    m, k, kt, dt = _M, _K, _KT, _DT
    din = fts.shape[1]
    xyzT = jnp.transpose(xyz, (0, 2, 1))

    xyz3 = jnp.transpose(xyz, (2, 0, 1))
    center_xyz = jnp.transpose(_fps(xyz3, m), (0, 2, 1))

    knn_ids = _knn(center_xyz, xyzT, k, tm=256)

    ftsT = jnp.transpose(fts, (0, 2, 1))
    w1f, w1x = W1[:, :din], W1[:, din:]
    f1 = _f1(ftsT, xyz, w1f, w1x)
    f1g = _sc_gather(f1.reshape(b * n, -1), _flat_ids(knn_ids, b, n))
    f1g = f1g.reshape(b, m * k, -1)

    x = _mlp(f1g, center_xyz, w1x, b1.reshape(1, -1), W2, b2.reshape(1, -1),
             k, tm=128)

    wkv = jnp.concatenate([Wk, Wv], axis=0)
    q, t2 = _qt2(x, center_xyz, Wq, wkv, Wp1)
    cwq = t2[..., 2 * dt:3 * dt]

    cT = jnp.transpose(center_xyz, (0, 2, 1))
    nb = _knn(center_xyz, cT, kt, tm=256)

    t2g = _sc_gather(t2.reshape(b * m, -1), _flat_ids(nb, b, m))
    t2g = t2g.reshape(b, m * kt, -1)

    y = _attn(q, t2g, cwq, x,
              bp1.reshape(1, -1), Wp2, bp2.reshape(1, -1),
              Wg1, bg1.reshape(1, -1), Wg2, bg2.reshape(1, -1),
              Wo, bo.reshape(1, -1), kt, tm=256)

    center_fts = jnp.transpose(y, (0, 2, 1))
    return center_xyz, center_fts


# speedup vs baseline: 1.9355x; 1.0196x over previous
import functools

import jax
import jax.numpy as jnp
from jax import lax
from jax.experimental import pallas as pl
from jax.experimental.pallas import tpu as pltpu
from jax.experimental.pallas import tpu_sc as plsc

_M, _K, _KT, _DT = 1024, 32, 16, 64


def _fps_body(m, xyz3_ref, ctr_ref, d2_ref):
    _, b, n = xyz3_ref.shape
    s = xyz3_ref[...].reshape(3 * b, n)
    lane3 = jax.lax.broadcasted_iota(jnp.int32, (3 * b, n), 1)
    mlane = jax.lax.broadcasted_iota(jnp.int32, (b, m), 1)
    inf = jnp.float32(jnp.inf)
    d2_ref[...] = jnp.full((b, n), inf, jnp.float32)
    zero = jnp.zeros((b, m), jnp.float32)

    def extract(cur):
        sel3 = lane3 == jnp.broadcast_to(cur[None], (3, b, 1)).reshape(3 * b, 1)
        return jnp.max(jnp.where(sel3, s, -inf), axis=1, keepdims=True)

    def body(i, carry):
        cx, cy, cz, cur = carry
        l3 = extract(cur)
        rec = mlane == (i - 1)
        cx = jnp.where(rec, l3[:b], cx)
        cy = jnp.where(rec, l3[b:2 * b], cy)
        cz = jnp.where(rec, l3[2 * b:], cz)
        d3 = s - l3
        sq = d3 * d3
        dist = sq[:b] + sq[b:2 * b] + sq[2 * b:]
        d2 = jnp.minimum(d2_ref[...], dist)
        d2_ref[...] = d2
        nxt = jnp.argmax(d2, axis=1).astype(jnp.int32)[:, None]
        return cx, cy, cz, nxt

    init = (zero, zero, zero, jnp.zeros((b, 1), jnp.int32))
    cx, cy, cz, cur = jax.lax.fori_loop(1, m, body, init)
    l3 = extract(cur)
    rec = mlane == (m - 1)
    ctr_ref[:, 0, :] = jnp.where(rec, l3[:b], cx)
    ctr_ref[:, 1, :] = jnp.where(rec, l3[b:2 * b], cy)
    ctr_ref[:, 2, :] = jnp.where(rec, l3[2 * b:], cz)


def _fps(xyz3, m):
    _, b, n = xyz3.shape
    return pl.pallas_call(
        functools.partial(_fps_body, m),
        out_shape=jax.ShapeDtypeStruct((b, 3, m), jnp.float32),
        scratch_shapes=[pltpu.VMEM((b, n), jnp.float32)],
    )(xyz3)


def _sc_gather(table, idx, ch=128):
    r, d = table.shape
    bsz = idx.shape[0]
    nw = 32
    b_per_w = bsz // nw
    ch = min(ch, b_per_w)
    nch = b_per_w // ch
    assert b_per_w % ch == 0 and nch % 2 == 0 or nch == 1, (bsz, ch)
    mesh = plsc.VectorSubcoreMesh(core_axis_name="c", subcore_axis_name="s")

    @functools.partial(
        pl.kernel, mesh=mesh,
        out_type=jax.ShapeDtypeStruct((bsz, d), table.dtype),
        scratch_types=[
            pltpu.VMEM((2, ch), jnp.int32),
            pltpu.VMEM((2, ch, d), table.dtype),
            pltpu.SemaphoreType.DMA,
            pltpu.SemaphoreType.DMA,
            pltpu.SemaphoreType.DMA,
            pltpu.SemaphoreType.DMA,
            pltpu.SemaphoreType.DMA,
            pltpu.SemaphoreType.DMA,
        ])
    def k(table_hbm, idx_hbm, out_hbm, idx_v, rows_v,
          si0, si1, sg0, sg1, so0, so1):
        wid = lax.axis_index("s") * 2 + lax.axis_index("c")
        base = wid * b_per_w
        si = (si0, si1)
        sg = (sg0, sg1)
        so = (so0, so1)

        if nch == 1:
            pltpu.sync_copy(idx_hbm.at[pl.ds(base, ch)], idx_v.at[0])
            pltpu.async_copy(table_hbm.at[idx_v.at[0]], rows_v.at[0],
                             sg0).wait()
            pltpu.sync_copy(rows_v.at[0], out_hbm.at[pl.ds(base, ch)])
            return

        def step(s, _):
            c0 = base + (2 * s) * ch
            c1 = c0 + ch
            cp_i0 = pltpu.async_copy(idx_hbm.at[pl.ds(c0, ch)],
                                     idx_v.at[0], si[0])
            cp_i1 = pltpu.async_copy(idx_hbm.at[pl.ds(c1, ch)],
                                     idx_v.at[1], si[1])
            cp_i0.wait()
            cp_g0 = pltpu.async_copy(table_hbm.at[idx_v.at[0]],
                                     rows_v.at[0], sg[0])
            cp_i1.wait()
            cp_g0.wait()
            cp_g1 = pltpu.async_copy(table_hbm.at[idx_v.at[1]],
                                     rows_v.at[1], sg[1])
            cp_o0 = pltpu.async_copy(rows_v.at[0],
                                     out_hbm.at[pl.ds(c0, ch)], so[0])
            cp_g1.wait()
            cp_o1 = pltpu.async_copy(rows_v.at[1],
                                     out_hbm.at[pl.ds(c1, ch)], so[1])
            cp_o0.wait()
            cp_o1.wait()
            return _

        jax.lax.fori_loop(0, nch // 2, step, 0)

    return k(table, idx)


def _flat_ids(ids, b, stride):
    off = (jnp.arange(b, dtype=jnp.int32) * stride).reshape(
        (b,) + (1,) * (ids.ndim - 1))
    return (ids + off).reshape(-1)


def _knn_body(k, q_ref, rT_ref, ids_ref, d2_ref):
    tm = q_ref.shape[1]
    n = rT_ref.shape[2]
    q = q_ref[0]
    rT = rT_ref[0]
    qq = jnp.sum(q * q, axis=1, keepdims=True)
    rr = jnp.sum(rT * rT, axis=0, keepdims=True)
    cross = jax.lax.dot_general(
        q, rT, (((1,), (0,)), ((), ())), preferred_element_type=jnp.float32)
    d2_ref[...] = qq - 2.0 * cross + rr
    lane = jax.lax.broadcasted_iota(jnp.int32, (tm, n), 1)
    klane = jax.lax.broadcasted_iota(jnp.int32, (tm, k), 1)
    inf = jnp.float32(jnp.inf)

    def body(j, ids_acc):
        d2 = d2_ref[...]
        mn = jnp.min(d2, axis=1, keepdims=True)
        am = jnp.min(jnp.where(d2 == mn, lane, n), axis=1, keepdims=True)
        ids_acc = jnp.where(klane == j, am, ids_acc)
        d2_ref[...] = jnp.where(lane == am, inf, d2)
        return ids_acc

    ids_ref[0] = jax.lax.fori_loop(
        0, k, body, jnp.zeros((tm, k), jnp.int32))


def _knn(q, rT, k, tm):
    b, mq, _ = q.shape
    n = rT.shape[2]
    return pl.pallas_call(
        functools.partial(_knn_body, k),
        grid=(b, mq // tm),
        in_specs=[
            pl.BlockSpec((1, tm, 3), lambda i, j: (i, j, 0)),
            pl.BlockSpec((1, 3, n), lambda i, j: (i, 0, 0)),
        ],
        out_specs=pl.BlockSpec((1, tm, k), lambda i, j: (i, j, 0)),
        out_shape=jax.ShapeDtypeStruct((b, mq, k), jnp.int32),
        scratch_shapes=[pltpu.VMEM((tm, n), jnp.float32)],
    )(q, rT)


def _f1_body(a_ref, w1f_ref, c_ref, w1x_ref, out_ref):
    out_ref[0] = (
        jax.lax.dot_general(a_ref[0], w1f_ref[...], (((1,), (1,)), ((), ())),
                            preferred_element_type=jnp.float32)
        + jax.lax.dot_general(c_ref[0], w1x_ref[...], (((1,), (1,)), ((), ())),
                              preferred_element_type=jnp.float32))


def _f1(ftsT, xyz, w1f, w1x):
    b, n, c = ftsT.shape
    o = w1f.shape[0]
    return pl.pallas_call(
        _f1_body,
        grid=(b,),
        in_specs=[
            pl.BlockSpec((1, n, c), lambda i: (i, 0, 0)),
            pl.BlockSpec(w1f.shape, lambda i: (0, 0)),
            pl.BlockSpec((1, n, 3), lambda i: (i, 0, 0)),
            pl.BlockSpec(w1x.shape, lambda i: (0, 0)),
        ],
        out_specs=pl.BlockSpec((1, n, o), lambda i: (i, 0, 0)),
        out_shape=jax.ShapeDtypeStruct((b, n, o), jnp.float32),
    )(ftsT, w1f, xyz, w1x)


def _mlp_body(k, f1g_ref, c_ref, w1x_ref, b1_ref, w2_ref, b2_ref, out_ref):
    rows = f1g_ref.shape[1]
    tm = rows // k
    d1 = f1g_ref.shape[2]
    cterm = jax.lax.dot_general(
        c_ref[0], w1x_ref[...], (((1,), (1,)), ((), ())),
        preferred_element_type=jnp.float32)
    crep = jnp.broadcast_to(cterm[:, None, :], (tm, k, d1)).reshape(rows, d1)
    z = jnp.maximum(f1g_ref[0].astype(jnp.float32) - crep + b1_ref[...], 0.0)
    y = jax.lax.dot_general(
        z, w2_ref[...], (((1,), (1,)), ((), ())),
        preferred_element_type=jnp.float32) + b2_ref[...]
    d2o = y.shape[1]
    out_ref[0] = jnp.max(y.reshape(tm, k, d2o), axis=1)


def _mlp(f1g, c, w1x, b1, w2, b2, k, tm):
    b, rows, d1 = f1g.shape
    m = rows // k
    d2o = w2.shape[0]
    grid = (b, m // tm)
    return pl.pallas_call(
        functools.partial(_mlp_body, k),
        grid=grid,
        in_specs=[
            pl.BlockSpec((1, tm * k, d1), lambda i, j: (i, j, 0)),
            pl.BlockSpec((1, tm, 3), lambda i, j: (i, j, 0)),
            pl.BlockSpec(w1x.shape, lambda i, j: (0, 0)),
            pl.BlockSpec(b1.shape, lambda i, j: (0, 0)),
            pl.BlockSpec(w2.shape, lambda i, j: (0, 0)),
            pl.BlockSpec(b2.shape, lambda i, j: (0, 0)),
        ],
        out_specs=pl.BlockSpec((1, tm, d2o), lambda i, j: (i, j, 0)),
        out_shape=jax.ShapeDtypeStruct((b, m, d2o), jnp.float32),
    )(f1g, c, w1x, b1, w2, b2)


def _qt2_body(x_ref, c_ref, wq_ref, wkv_ref, wp1_ref, q_ref, t2_ref):
    x = x_ref[0]
    q_ref[0] = jax.lax.dot_general(
        x, wq_ref[...], (((1,), (1,)), ((), ())),
        preferred_element_type=jnp.float32)
    kv = jax.lax.dot_general(
        x, wkv_ref[...], (((1,), (1,)), ((), ())),
        preferred_element_type=jnp.float32)
    cw = jax.lax.dot_general(
        c_ref[0], wp1_ref[...], (((1,), (1,)), ((), ())),
        preferred_element_type=jnp.float32)
    t2_ref[0] = jnp.concatenate([kv, cw, cw], axis=1)


def _qt2(x, c, wq, wkv, wp1):
    b, m, _ = x.shape
    dt = wq.shape[0]
    return pl.pallas_call(
        _qt2_body,
        grid=(b,),
        in_specs=[
            pl.BlockSpec((1, m, x.shape[2]), lambda i: (i, 0, 0)),
            pl.BlockSpec((1, m, 3), lambda i: (i, 0, 0)),
            pl.BlockSpec(wq.shape, lambda i: (0, 0)),
            pl.BlockSpec(wkv.shape, lambda i: (0, 0)),
            pl.BlockSpec(wp1.shape, lambda i: (0, 0)),
        ],
        out_specs=[
            pl.BlockSpec((1, m, dt), lambda i: (i, 0, 0)),
            pl.BlockSpec((1, m, 4 * dt), lambda i: (i, 0, 0)),
        ],
        out_shape=[
            jax.ShapeDtypeStruct((b, m, dt), jnp.float32),
            jax.ShapeDtypeStruct((b, m, 4 * dt), jnp.float32),
        ],
    )(x, c, wq, wkv, wp1)


def _attn_body(kt, q_ref, t2g_ref, cwq_ref, x_ref,
               bp1_ref, wp2_ref, bp2_ref,
               wg1_ref, bg1_ref, wg2_ref, bg2_ref,
               wo_ref, bo_ref, out_ref):
    rows = t2g_ref.shape[1]
    tm = rows // kt
    dt = q_ref.shape[2]
    t2 = t2g_ref[0].astype(jnp.float32)
    kg = t2[:, :dt]
    vg = t2[:, dt:2 * dt]
    cwg = t2[:, 2 * dt:3 * dt]
    cwq = cwq_ref[0].astype(jnp.float32)
    cwq_rep = jnp.broadcast_to(
        cwq[:, None, :], (tm, kt, dt)).reshape(rows, dt)
    delta = jnp.maximum(cwq_rep - cwg + bp1_ref[...], 0.0)
    delta = jax.lax.dot_general(
        delta, wp2_ref[...], (((1,), (1,)), ((), ())),
        preferred_element_type=jnp.float32) + bp2_ref[...]
    q = q_ref[0]
    qrep = jnp.broadcast_to(q[:, None, :], (tm, kt, dt)).reshape(rows, dt)
    a = qrep - kg + delta
    a = jax.lax.dot_general(
        a, wg1_ref[...], (((1,), (1,)), ((), ())),
        preferred_element_type=jnp.float32) + bg1_ref[...]
    a = jnp.maximum(a, 0.0)
    a = jax.lax.dot_general(
        a, wg2_ref[...], (((1,), (1,)), ((), ())),
        preferred_element_type=jnp.float32) + bg2_ref[...]
    a3 = a.reshape(tm, kt, dt)
    amax = jnp.max(a3, axis=1, keepdims=True)
    e = jnp.exp(a3 - amax)
    s = jnp.sum(e, axis=1, keepdims=True)
    attn = e / s
    vpd = (vg + delta).reshape(tm, kt, dt)
    y = jnp.sum(attn * vpd, axis=1)
    out = jax.lax.dot_general(
        y, wo_ref[...], (((1,), (1,)), ((), ())),
        preferred_element_type=jnp.float32) + bo_ref[...]
    out_ref[0] = out + x_ref[0]


def _attn(q, t2g, cwq, x, bp1, wp2, bp2, wg1, bg1, wg2, bg2, wo, bo, kt, tm):
    b, m, dt = q.shape
    do = wo.shape[0]
    grid = (b, m // tm)
    return pl.pallas_call(
        functools.partial(_attn_body, kt),
        grid=grid,
        in_specs=[
            pl.BlockSpec((1, tm, dt), lambda i, j: (i, j, 0)),
            pl.BlockSpec((1, tm * kt, 4 * dt), lambda i, j: (i, j, 0)),
            pl.BlockSpec((1, tm, dt), lambda i, j: (i, j, 0)),
            pl.BlockSpec((1, tm, do), lambda i, j: (i, j, 0)),
            pl.BlockSpec(bp1.shape, lambda i, j: (0, 0)),
            pl.BlockSpec(wp2.shape, lambda i, j: (0, 0)),
            pl.BlockSpec(bp2.shape, lambda i, j: (0, 0)),
            pl.BlockSpec(wg1.shape, lambda i, j: (0, 0)),
            pl.BlockSpec(bg1.shape, lambda i, j: (0, 0)),
            pl.BlockSpec(wg2.shape, lambda i, j: (0, 0)),
            pl.BlockSpec(bg2.shape, lambda i, j: (0, 0)),
            pl.BlockSpec(wo.shape, lambda i, j: (0, 0)),
            pl.BlockSpec(bo.shape, lambda i, j: (0, 0)),
        ],
        out_specs=pl.BlockSpec((1, tm, do), lambda i, j: (i, j, 0)),
        out_shape=jax.ShapeDtypeStruct((b, m, do), jnp.float32),
    )(q, t2g, cwq, x, bp1, wp2, bp2, wg1, bg1, wg2, bg2, wo, bo)


def kernel(xyz, fts, W1, b1, W2, b2, Wq, Wk, Wv, Wp1, bp1, Wp2, bp2,
           Wg1, bg1, Wg2, bg2, Wo, bo):
    b, n, _ = xyz.shape
    m, k, kt, dt = _M, _K, _KT, _DT
    din = fts.shape[1]
    xyzT = jnp.transpose(xyz, (0, 2, 1))

    xyz3 = jnp.transpose(xyz, (2, 0, 1))
    center_xyz = jnp.transpose(_fps(xyz3, m), (0, 2, 1))

    knn_ids = _knn(center_xyz, xyzT, k, tm=512)

    ftsT = jnp.transpose(fts, (0, 2, 1))
    w1f, w1x = W1[:, :din], W1[:, din:]
    f1 = _f1(ftsT, xyz, w1f, w1x)
    f1g = _sc_gather(f1.reshape(b * n, -1), _flat_ids(knn_ids, b, n))
    f1g = f1g.reshape(b, m * k, -1)

    x = _mlp(f1g, center_xyz, w1x, b1.reshape(1, -1), W2, b2.reshape(1, -1),
             k, tm=128)

    wkv = jnp.concatenate([Wk, Wv], axis=0)
    q, t2 = _qt2(x, center_xyz, Wq, wkv, Wp1)
    cwq = t2[..., 2 * dt:3 * dt]

    cT = jnp.transpose(center_xyz, (0, 2, 1))
    nb = _knn(center_xyz, cT, kt, tm=256)

    t2g = _sc_gather(t2.reshape(b * m, -1), _flat_ids(nb, b, m))
    t2g = t2g.reshape(b, m * kt, -1)

    y = _attn(q, t2g, cwq, x,
              bp1.reshape(1, -1), Wp2, bp2.reshape(1, -1),
              Wg1, bg1.reshape(1, -1), Wg2, bg2.reshape(1, -1),
              Wo, bo.reshape(1, -1), kt, tm=256)

    center_fts = jnp.transpose(y, (0, 2, 1))
    return center_xyz, center_fts


# speedup vs baseline: 1.9385x; 1.0015x over previous
import functools

import jax
import jax.numpy as jnp
from jax import lax
from jax.experimental import pallas as pl
from jax.experimental.pallas import tpu as pltpu
from jax.experimental.pallas import tpu_sc as plsc

_M, _K, _KT, _DT = 1024, 32, 16, 64


def _fps_body(m, xyz3_ref, ctr_ref, d2_ref):
    _, b, n = xyz3_ref.shape
    s = xyz3_ref[...].reshape(3 * b, n)
    lane3 = jax.lax.broadcasted_iota(jnp.int32, (3 * b, n), 1)
    mlane = jax.lax.broadcasted_iota(jnp.int32, (b, m), 1)
    inf = jnp.float32(jnp.inf)
    d2_ref[...] = jnp.full((b, n), inf, jnp.float32)
    zero = jnp.zeros((b, m), jnp.float32)

    def extract(cur):
        sel3 = lane3 == jnp.broadcast_to(cur[None], (3, b, 1)).reshape(3 * b, 1)
        return jnp.max(jnp.where(sel3, s, -inf), axis=1, keepdims=True)

    def body(i, carry):
        cx, cy, cz, cur = carry
        l3 = extract(cur)
        rec = mlane == (i - 1)
        cx = jnp.where(rec, l3[:b], cx)
        cy = jnp.where(rec, l3[b:2 * b], cy)
        cz = jnp.where(rec, l3[2 * b:], cz)
        d3 = s - l3
        sq = d3 * d3
        dist = sq[:b] + sq[b:2 * b] + sq[2 * b:]
        d2 = jnp.minimum(d2_ref[...], dist)
        d2_ref[...] = d2
        nxt = jnp.argmax(d2, axis=1).astype(jnp.int32)[:, None]
        return cx, cy, cz, nxt

    init = (zero, zero, zero, jnp.zeros((b, 1), jnp.int32))
    cx, cy, cz, cur = jax.lax.fori_loop(1, m, body, init)
    l3 = extract(cur)
    rec = mlane == (m - 1)
    ctr_ref[:, 0, :] = jnp.where(rec, l3[:b], cx)
    ctr_ref[:, 1, :] = jnp.where(rec, l3[b:2 * b], cy)
    ctr_ref[:, 2, :] = jnp.where(rec, l3[2 * b:], cz)


def _fps(xyz3, m):
    _, b, n = xyz3.shape
    return pl.pallas_call(
        functools.partial(_fps_body, m),
        out_shape=jax.ShapeDtypeStruct((b, 3, m), jnp.float32),
        scratch_shapes=[pltpu.VMEM((b, n), jnp.float32)],
    )(xyz3)


def _sc_gather(table, idx, ch=128):
    r, d = table.shape
    bsz = idx.shape[0]
    nw = 32
    b_per_w = bsz // nw
    ch = min(ch, b_per_w)
    nch = b_per_w // ch
    assert b_per_w % ch == 0 and nch % 2 == 0 or nch == 1, (bsz, ch)
    mesh = plsc.VectorSubcoreMesh(core_axis_name="c", subcore_axis_name="s")

    @functools.partial(
        pl.kernel, mesh=mesh,
        out_type=jax.ShapeDtypeStruct((bsz, d), table.dtype),
        scratch_types=[
            pltpu.VMEM((2, ch), jnp.int32),
            pltpu.VMEM((2, ch, d), table.dtype),
            pltpu.SemaphoreType.DMA,
            pltpu.SemaphoreType.DMA,
            pltpu.SemaphoreType.DMA,
            pltpu.SemaphoreType.DMA,
            pltpu.SemaphoreType.DMA,
            pltpu.SemaphoreType.DMA,
        ])
    def k(table_hbm, idx_hbm, out_hbm, idx_v, rows_v,
          si0, si1, sg0, sg1, so0, so1):
        wid = lax.axis_index("s") * 2 + lax.axis_index("c")
        base = wid * b_per_w
        si = (si0, si1)
        sg = (sg0, sg1)
        so = (so0, so1)

        if nch == 1:
            pltpu.sync_copy(idx_hbm.at[pl.ds(base, ch)], idx_v.at[0])
            pltpu.async_copy(table_hbm.at[idx_v.at[0]], rows_v.at[0],
                             sg0).wait()
            pltpu.sync_copy(rows_v.at[0], out_hbm.at[pl.ds(base, ch)])
            return

        def step(s, _):
            c0 = base + (2 * s) * ch
            c1 = c0 + ch
            cp_i0 = pltpu.async_copy(idx_hbm.at[pl.ds(c0, ch)],
                                     idx_v.at[0], si[0])
            cp_i1 = pltpu.async_copy(idx_hbm.at[pl.ds(c1, ch)],
                                     idx_v.at[1], si[1])
            cp_i0.wait()
            cp_g0 = pltpu.async_copy(table_hbm.at[idx_v.at[0]],
                                     rows_v.at[0], sg[0])
            cp_i1.wait()
            cp_g0.wait()
            cp_g1 = pltpu.async_copy(table_hbm.at[idx_v.at[1]],
                                     rows_v.at[1], sg[1])
            cp_o0 = pltpu.async_copy(rows_v.at[0],
                                     out_hbm.at[pl.ds(c0, ch)], so[0])
            cp_g1.wait()
            cp_o1 = pltpu.async_copy(rows_v.at[1],
                                     out_hbm.at[pl.ds(c1, ch)], so[1])
            cp_o0.wait()
            cp_o1.wait()
            return _

        jax.lax.fori_loop(0, nch // 2, step, 0)

    return k(table, idx)


def _flat_ids(ids, b, stride):
    off = (jnp.arange(b, dtype=jnp.int32) * stride).reshape(
        (b,) + (1,) * (ids.ndim - 1))
    return (ids + off).reshape(-1)


def _knn_body(k, q_ref, rT_ref, ids_ref, d2_ref):
    tm = q_ref.shape[1]
    n = rT_ref.shape[2]
    q = q_ref[0]
    rT = rT_ref[0]
    qq = jnp.sum(q * q, axis=1, keepdims=True)
    rr = jnp.sum(rT * rT, axis=0, keepdims=True)
    cross = jax.lax.dot_general(
        q, rT, (((1,), (0,)), ((), ())), preferred_element_type=jnp.float32)
    d2_ref[...] = qq - 2.0 * cross + rr
    lane = jax.lax.broadcasted_iota(jnp.int32, (tm, n), 1)
    klane = jax.lax.broadcasted_iota(jnp.int32, (tm, k), 1)
    inf = jnp.float32(jnp.inf)

    def body(j, ids_acc):
        d2 = d2_ref[...]
        mn = jnp.min(d2, axis=1, keepdims=True)
        am = jnp.min(jnp.where(d2 == mn, lane, n), axis=1, keepdims=True)
        ids_acc = jnp.where(klane == j, am, ids_acc)
        d2_ref[...] = jnp.where(lane == am, inf, d2)
        return ids_acc

    ids_ref[0] = jax.lax.fori_loop(
        0, k, body, jnp.zeros((tm, k), jnp.int32))


def _knn(q, rT, k, tm):
    b, mq, _ = q.shape
    n = rT.shape[2]
    return pl.pallas_call(
        functools.partial(_knn_body, k),
        grid=(b, mq // tm),
        in_specs=[
            pl.BlockSpec((1, tm, 3), lambda i, j: (i, j, 0)),
            pl.BlockSpec((1, 3, n), lambda i, j: (i, 0, 0)),
        ],
        out_specs=pl.BlockSpec((1, tm, k), lambda i, j: (i, j, 0)),
        out_shape=jax.ShapeDtypeStruct((b, mq, k), jnp.int32),
        scratch_shapes=[pltpu.VMEM((tm, n), jnp.float32)],
    )(q, rT)


def _f1_body(a_ref, w1f_ref, c_ref, w1x_ref, out_ref):
    out_ref[0] = (
        jax.lax.dot_general(a_ref[0], w1f_ref[...], (((1,), (1,)), ((), ())),
                            preferred_element_type=jnp.float32)
        + jax.lax.dot_general(c_ref[0], w1x_ref[...], (((1,), (1,)), ((), ())),
                              preferred_element_type=jnp.float32))


def _f1(ftsT, xyz, w1f, w1x):
    b, n, c = ftsT.shape
    o = w1f.shape[0]
    return pl.pallas_call(
        _f1_body,
        grid=(b,),
        in_specs=[
            pl.BlockSpec((1, n, c), lambda i: (i, 0, 0)),
            pl.BlockSpec(w1f.shape, lambda i: (0, 0)),
            pl.BlockSpec((1, n, 3), lambda i: (i, 0, 0)),
            pl.BlockSpec(w1x.shape, lambda i: (0, 0)),
        ],
        out_specs=pl.BlockSpec((1, n, o), lambda i: (i, 0, 0)),
        out_shape=jax.ShapeDtypeStruct((b, n, o), jnp.float32),
    )(ftsT, w1f, xyz, w1x)


def _mlp_body(k, f1g_ref, c_ref, w1x_ref, b1_ref, w2_ref, b2_ref, out_ref):
    rows = f1g_ref.shape[1]
    tm = rows // k
    d1 = f1g_ref.shape[2]
    cterm = jax.lax.dot_general(
        c_ref[0], w1x_ref[...], (((1,), (1,)), ((), ())),
        preferred_element_type=jnp.float32)
    crep = jnp.broadcast_to(cterm[:, None, :], (tm, k, d1)).reshape(rows, d1)
    z = jnp.maximum(f1g_ref[0].astype(jnp.float32) - crep + b1_ref[...], 0.0)
    y = jax.lax.dot_general(
        z, w2_ref[...], (((1,), (1,)), ((), ())),
        preferred_element_type=jnp.float32) + b2_ref[...]
    d2o = y.shape[1]
    out_ref[0] = jnp.max(y.reshape(tm, k, d2o), axis=1)


def _mlp(f1g, c, w1x, b1, w2, b2, k, tm):
    b, rows, d1 = f1g.shape
    m = rows // k
    d2o = w2.shape[0]
    grid = (b, m // tm)
    return pl.pallas_call(
        functools.partial(_mlp_body, k),
        grid=grid,
        in_specs=[
            pl.BlockSpec((1, tm * k, d1), lambda i, j: (i, j, 0)),
            pl.BlockSpec((1, tm, 3), lambda i, j: (i, j, 0)),
            pl.BlockSpec(w1x.shape, lambda i, j: (0, 0)),
            pl.BlockSpec(b1.shape, lambda i, j: (0, 0)),
            pl.BlockSpec(w2.shape, lambda i, j: (0, 0)),
            pl.BlockSpec(b2.shape, lambda i, j: (0, 0)),
        ],
        out_specs=pl.BlockSpec((1, tm, d2o), lambda i, j: (i, j, 0)),
        out_shape=jax.ShapeDtypeStruct((b, m, d2o), jnp.float32),
    )(f1g, c, w1x, b1, w2, b2)


def _qt2_body(x_ref, c_ref, wq_ref, wkv_ref, wp1_ref, q_ref, t2_ref):
    x = x_ref[0]
    q_ref[0] = jax.lax.dot_general(
        x, wq_ref[...], (((1,), (1,)), ((), ())),
        preferred_element_type=jnp.float32)
    kv = jax.lax.dot_general(
        x, wkv_ref[...], (((1,), (1,)), ((), ())),
        preferred_element_type=jnp.float32)
    cw = jax.lax.dot_general(
        c_ref[0], wp1_ref[...], (((1,), (1,)), ((), ())),
        preferred_element_type=jnp.float32)
    t2_ref[0] = jnp.concatenate([kv, cw, cw], axis=1)


def _qt2(x, c, wq, wkv, wp1):
    b, m, _ = x.shape
    dt = wq.shape[0]
    return pl.pallas_call(
        _qt2_body,
        grid=(b,),
        in_specs=[
            pl.BlockSpec((1, m, x.shape[2]), lambda i: (i, 0, 0)),
            pl.BlockSpec((1, m, 3), lambda i: (i, 0, 0)),
            pl.BlockSpec(wq.shape, lambda i: (0, 0)),
            pl.BlockSpec(wkv.shape, lambda i: (0, 0)),
            pl.BlockSpec(wp1.shape, lambda i: (0, 0)),
        ],
        out_specs=[
            pl.BlockSpec((1, m, dt), lambda i: (i, 0, 0)),
            pl.BlockSpec((1, m, 4 * dt), lambda i: (i, 0, 0)),
        ],
        out_shape=[
            jax.ShapeDtypeStruct((b, m, dt), jnp.float32),
            jax.ShapeDtypeStruct((b, m, 4 * dt), jnp.float32),
        ],
    )(x, c, wq, wkv, wp1)


def _attn_body(kt, q_ref, t2g_ref, cwq_ref, x_ref,
               bp1_ref, wp2_ref, bp2_ref,
               wg1_ref, bg1_ref, wg2_ref, bg2_ref,
               wo_ref, bo_ref, out_ref):
    rows = t2g_ref.shape[1]
    tm = rows // kt
    dt = q_ref.shape[2]
    t2 = t2g_ref[0].astype(jnp.float32)
    kg = t2[:, :dt]
    vg = t2[:, dt:2 * dt]
    cwg = t2[:, 2 * dt:3 * dt]
    cwq = cwq_ref[0].astype(jnp.float32)
    cwq_rep = jnp.broadcast_to(
        cwq[:, None, :], (tm, kt, dt)).reshape(rows, dt)
    delta = jnp.maximum(cwq_rep - cwg + bp1_ref[...], 0.0)
    delta = jax.lax.dot_general(
        delta, wp2_ref[...], (((1,), (1,)), ((), ())),
        preferred_element_type=jnp.float32) + bp2_ref[...]
    q = q_ref[0]
    qrep = jnp.broadcast_to(q[:, None, :], (tm, kt, dt)).reshape(rows, dt)
    a = qrep - kg + delta
    a = jax.lax.dot_general(
        a, wg1_ref[...], (((1,), (1,)), ((), ())),
        preferred_element_type=jnp.float32) + bg1_ref[...]
    a = jnp.maximum(a, 0.0)
    a = jax.lax.dot_general(
        a, wg2_ref[...], (((1,), (1,)), ((), ())),
        preferred_element_type=jnp.float32) + bg2_ref[...]
    a3 = a.reshape(tm, kt, dt)
    amax = jnp.max(a3, axis=1, keepdims=True)
    e = jnp.exp(a3 - amax)
    s = jnp.sum(e, axis=1, keepdims=True)
    attn = e / s
    vpd = (vg + delta).reshape(tm, kt, dt)
    y = jnp.sum(attn * vpd, axis=1)
    out = jax.lax.dot_general(
        y, wo_ref[...], (((1,), (1,)), ((), ())),
        preferred_element_type=jnp.float32) + bo_ref[...]
    out_ref[0] = out + x_ref[0]


def _attn(q, t2g, cwq, x, bp1, wp2, bp2, wg1, bg1, wg2, bg2, wo, bo, kt, tm):
    b, m, dt = q.shape
    do = wo.shape[0]
    grid = (b, m // tm)
    return pl.pallas_call(
        functools.partial(_attn_body, kt),
        grid=grid,
        in_specs=[
            pl.BlockSpec((1, tm, dt), lambda i, j: (i, j, 0)),
            pl.BlockSpec((1, tm * kt, 4 * dt), lambda i, j: (i, j, 0)),
            pl.BlockSpec((1, tm, dt), lambda i, j: (i, j, 0)),
            pl.BlockSpec((1, tm, do), lambda i, j: (i, j, 0)),
            pl.BlockSpec(bp1.shape, lambda i, j: (0, 0)),
            pl.BlockSpec(wp2.shape, lambda i, j: (0, 0)),
            pl.BlockSpec(bp2.shape, lambda i, j: (0, 0)),
            pl.BlockSpec(wg1.shape, lambda i, j: (0, 0)),
            pl.BlockSpec(bg1.shape, lambda i, j: (0, 0)),
            pl.BlockSpec(wg2.shape, lambda i, j: (0, 0)),
            pl.BlockSpec(bg2.shape, lambda i, j: (0, 0)),
            pl.BlockSpec(wo.shape, lambda i, j: (0, 0)),
            pl.BlockSpec(bo.shape, lambda i, j: (0, 0)),
        ],
        out_specs=pl.BlockSpec((1, tm, do), lambda i, j: (i, j, 0)),
        out_shape=jax.ShapeDtypeStruct((b, m, do), jnp.float32),
    )(q, t2g, cwq, x, bp1, wp2, bp2, wg1, bg1, wg2, bg2, wo, bo)


def kernel(xyz, fts, W1, b1, W2, b2, Wq, Wk, Wv, Wp1, bp1, Wp2, bp2,
           Wg1, bg1, Wg2, bg2, Wo, bo):
    b, n, _ = xyz.shape
    m, k, kt, dt = _M, _K, _KT, _DT
    din = fts.shape[1]
    xyzT = jnp.transpose(xyz, (0, 2, 1))

    xyz3 = jnp.transpose(xyz, (2, 0, 1))
    center_xyz = jnp.transpose(_fps(xyz3, m), (0, 2, 1))

    knn_ids = _knn(center_xyz, xyzT, k, tm=1024)

    ftsT = jnp.transpose(fts, (0, 2, 1))
    w1f, w1x = W1[:, :din], W1[:, din:]
    f1 = _f1(ftsT, xyz, w1f, w1x)
    f1g = _sc_gather(f1.reshape(b * n, -1), _flat_ids(knn_ids, b, n))
    f1g = f1g.reshape(b, m * k, -1)

    x = _mlp(f1g, center_xyz, w1x, b1.reshape(1, -1), W2, b2.reshape(1, -1),
             k, tm=128)

    wkv = jnp.concatenate([Wk, Wv], axis=0)
    q, t2 = _qt2(x, center_xyz, Wq, wkv, Wp1)
    cwq = t2[..., 2 * dt:3 * dt]

    cT = jnp.transpose(center_xyz, (0, 2, 1))
    nb = _knn(center_xyz, cT, kt, tm=256)

    t2g = _sc_gather(t2.reshape(b * m, -1), _flat_ids(nb, b, m))
    t2g = t2g.reshape(b, m * kt, -1)

    y = _attn(q, t2g, cwq, x,
              bp1.reshape(1, -1), Wp2, bp2.reshape(1, -1),
              Wg1, bg1.reshape(1, -1), Wg2, bg2.reshape(1, -1),
              Wo, bo.reshape(1, -1), kt, tm=256)

    center_fts = jnp.transpose(y, (0, 2, 1))
    return center_xyz, center_fts
